# baseline (speedup 1.0000x reference)
import functools
import jax
import jax.numpy as jnp
from jax import lax
from jax.experimental import pallas as pl
from jax.experimental.pallas import tpu as pltpu
from jax.experimental.pallas import tpu_sc as plsc

H = 128
NC = 2
NS = 16
L = 16
CH = 128
ACCR = 10240


def _silu(x):
    return x * jax.nn.sigmoid(x)


def _round_up(n, m):
    return (n + m - 1) // m * m


def _proj_body(x_ref, w_ref, *out_refs):
    y = jnp.dot(x_ref[...], w_ref[...], preferred_element_type=jnp.float32)
    for i, o in enumerate(out_refs):
        o[...] = y[:, i * H:(i + 1) * H]


def _node_proj(x, wcat):
    n = x.shape[0]
    nout = wcat.shape[1] // H
    blk = 1000
    return pl.pallas_call(
        _proj_body,
        grid=(n // blk,),
        in_specs=[pl.BlockSpec((blk, H), lambda i: (i, 0)),
                  pl.BlockSpec(wcat.shape, lambda i: (0, 0))],
        out_specs=[pl.BlockSpec((blk, H), lambda i: (i, 0))
                   for _ in range(nout)],
        out_shape=[jax.ShapeDtypeStruct((n, H), jnp.float32)
                   for _ in range(nout)],
    )(x, wcat)


def _gather_body(epad, a_hbm, b_hbm, i0_hbm, i1_hbm, g_hbm,
                 i0all, i1all, av0, bv0, av1, bv1, sg0, sg1, ss0, ss1):
    c = lax.axis_index("c")
    s = lax.axis_index("s")
    w = s * NC + c
    cpw = epad // (NC * NS * CH)
    tb = w * cpw
    pltpu.sync_copy(i0_hbm.at[pl.ds(tb * CH, cpw * CH)], i0all)
    pltpu.sync_copy(i1_hbm.at[pl.ds(tb * CH, cpw * CH)], i1all)
    avs = (av0, av1)
    bvs = (bv0, bv1)
    sgs = (sg0, sg1)
    sss = (ss0, ss1)

    def issue(j, p):
        pltpu.async_copy(a_hbm.at[i0all.at[pl.ds(j * CH, CH)]], avs[p], sgs[p])
        pltpu.async_copy(b_hbm.at[i1all.at[pl.ds(j * CH, CH)]], bvs[p], sgs[p])

    def wait_gather(p):
        pltpu.make_async_copy(a_hbm.at[i0all.at[pl.ds(0, CH)]],
                              avs[p], sgs[p]).wait()
        pltpu.make_async_copy(b_hbm.at[i1all.at[pl.ds(0, CH)]],
                              bvs[p], sgs[p]).wait()

    def wait_store(p):
        pltpu.make_async_copy(avs[p], g_hbm.at[0, pl.ds(0, CH)],
                              sss[p]).wait()
        pltpu.make_async_copy(bvs[p], g_hbm.at[1, pl.ds(0, CH)],
                              sss[p]).wait()

    def process(j, p):
        wait_gather(p)
        base = (tb + j) * CH
        pltpu.async_copy(avs[p], g_hbm.at[0, pl.ds(base, CH)], sss[p])
        pltpu.async_copy(bvs[p], g_hbm.at[1, pl.ds(base, CH)], sss[p])

    issue(0, 0)
    if cpw > 1:
        issue(1, 1)
        process(0, 0)
    if cpw > 2:
        wait_store(0)
        issue(2, 0)
        process(1, 1)
    npairs = (cpw - 3) // 2

    def pair(t, carry):
        j = 3 + 2 * t
        wait_store(1)
        issue(j, 1)
        process(j - 1, 0)
        wait_store(0)
        issue(j + 1, 0)
        process(j, 1)
        return carry

    lax.fori_loop(0, npairs, pair, 0)
    rem = 3 + 2 * npairs
    if cpw > 2 and rem < cpw:
        wait_store(1)
        issue(rem, 1)
        process(rem - 1, 0)
        process(rem, 1)
    elif cpw > 2:
        process(cpw - 1, (cpw - 1) % 2)
    elif cpw == 2:
        process(1, 1)
    elif cpw == 1:
        process(0, 0)
    wait_store(0)
    if cpw > 1:
        wait_store(1)


def _gather_sum(a, b, i0p, i1p):
    epad = i0p.shape[0]
    cpw = epad // (NC * NS * CH)
    mesh = plsc.VectorSubcoreMesh(core_axis_name="c", subcore_axis_name="s")
    kfn = pl.kernel(
        functools.partial(_gather_body, epad),
        out_type=jax.ShapeDtypeStruct((2, epad, H), jnp.float32),
        mesh=mesh,
        scratch_types=[
            pltpu.VMEM((cpw * CH,), jnp.int32),
            pltpu.VMEM((cpw * CH,), jnp.int32),
            pltpu.VMEM((CH, H), jnp.float32),
            pltpu.VMEM((CH, H), jnp.float32),
            pltpu.VMEM((CH, H), jnp.float32),
            pltpu.VMEM((CH, H), jnp.float32),
            pltpu.SemaphoreType.DMA,
            pltpu.SemaphoreType.DMA,
            pltpu.SemaphoreType.DMA,
            pltpu.SemaphoreType.DMA,
        ],
    )
    return kfn(a, b, i0p, i1p)


def _edge_mlp_body(ninv, g_ref, inv_ref, winv_ref, b1_ref, w2_ref, b2_ref,
                   ew_ref, eb_ref, z_ref):
    pre = g_ref[0] + g_ref[1] + b1_ref[...]
    for k in range(ninv):
        pre = pre + inv_ref[:, k:k + 1] * winv_ref[k:k + 1, :]
    m = _silu(pre)
    m2 = _silu(jnp.dot(m, w2_ref[...], preferred_element_type=jnp.float32)
               + b2_ref[...])
    logit = jnp.sum(m2 * ew_ref[...], axis=1, keepdims=True) + eb_ref[0, 0]
    z_ref[...] = m2 * jax.nn.sigmoid(logit)


def _edge_mlp(g, invp, winv, b1, w2, b2, ewr, ebr, ninv):
    epad = g.shape[1]
    blk = 512
    return pl.pallas_call(
        functools.partial(_edge_mlp_body, ninv),
        grid=(epad // blk,),
        in_specs=[
            pl.BlockSpec((2, blk, H), lambda i: (0, i, 0)),
            pl.BlockSpec((blk, 8), lambda i: (i, 0)),
            pl.BlockSpec((8, H), lambda i: (0, 0)),
            pl.BlockSpec((1, H), lambda i: (0, 0)),
            pl.BlockSpec((H, H), lambda i: (0, 0)),
            pl.BlockSpec((1, H), lambda i: (0, 0)),
            pl.BlockSpec((1, H), lambda i: (0, 0)),
            pl.BlockSpec((1, H), lambda i: (0, 0)),
        ],
        out_specs=pl.BlockSpec((blk, H), lambda i: (i, 0)),
        out_shape=jax.ShapeDtypeStruct((epad, H), jnp.float32),
    )(g, invp, winv, b1, w2, b2, ewr, ebr)


ZR = 32


def _zero_fill(zrow):
    def zrowfill(r, carry):
        for k in range(H // L):
            zrow[r, pl.ds(k * L, L)] = jnp.zeros((L,), jnp.float32)
        return carry

    lax.fori_loop(0, ZR, zrowfill, 0)


def _zero_acc(acc, zrow, s):
    zpt = ACCR // NS

    def zchunk(q, carry):
        pltpu.sync_copy(zrow, acc.at[pl.ds(s * zpt + q * ZR, ZR)])
        return carry

    lax.fori_loop(0, zpt // ZR, zchunk, 0)


def _writeout(acc, out_hbm, s, nrows, out_base):
    nch = nrows // 8
    npt = (nch + NS - 1) // NS

    def wo(j, carry):
        ch = j * NS + s

        @pl.when(ch < nch)
        def _():
            pltpu.sync_copy(acc.at[pl.ds(ch * 8, 8)],
                            out_hbm.at[pl.ds(out_base + ch * 8, 8)])
        return carry

    lax.fori_loop(0, npt, wo, 0)


def _scatter_body(epad, nrec, esplit, z_hbm, idx_hbm, out_hbm,
                  acc, zv0, zv1, iv0, iv1, lbuf, zrow,
                  sl0, sl1, si0, si1, sa0, sa1):
    c = lax.axis_index("c")
    s = lax.axis_index("s")
    if esplit:
        half = nrec
        cpt = epad // (NC * NS * CH)
        tb = (c * NS + s) * cpt
        lo = c * 0
    else:
        half = nrec // 2
        cpt = epad // (NS * CH)
        tb = s * cpt
        lo = c * half
    _zero_fill(zrow)
    _zero_acc(acc, zrow, s)
    plsc.subcore_barrier()

    zvs = (zv0, zv1)
    ivs = (iv0, iv1)
    sls = (sl0, sl1)
    sis = (si0, si1)
    sas = (sa0, sa1)

    def zload(j, p):
        pltpu.async_copy(z_hbm.at[pl.ds((tb + j) * CH, CH)], zvs[p], sls[p])
        pltpu.async_copy(idx_hbm.at[pl.ds((tb + j) * CH, CH)], ivs[p], sis[p])

    def scat(j, p):
        pltpu.make_async_copy(idx_hbm.at[pl.ds(0, CH)], ivs[p],
                              sis[p]).wait()
        for k in range(CH // L):
            sl = pl.ds(k * L, L)
            t = ivs[p][sl] - lo
            ok = (t >= 0) & (t < half)
            lbuf[p, sl] = jnp.where(ok, t, half)
        pltpu.make_async_copy(z_hbm.at[pl.ds(0, CH)], zvs[p], sls[p]).wait()
        pltpu.async_copy(zvs[p], acc.at[lbuf.at[p]], sas[p], add=True)

    def wait_scat(j, p):
        pltpu.make_async_copy(zvs[p], acc.at[lbuf.at[p]], sas[p]).wait()

    zload(0, 0)
    if cpt > 1:
        zload(1, 1)
        scat(0, 0)
    npairs = (cpt - 2) // 2 if cpt > 2 else 0

    def pair(t, carry):
        j = 2 + 2 * t
        wait_scat(j - 2, 0)
        zload(j, 0)
        scat(j - 1, 1)
        wait_scat(j - 1, 1)
        zload(j + 1, 1)
        scat(j, 0)
        return carry

    lax.fori_loop(0, npairs, pair, 0)
    rem = 2 + 2 * npairs
    if cpt > 2 and rem < cpt:
        wait_scat(rem - 2, 0)
        zload(rem, 0)
        scat(rem - 1, 1)
        scat(rem, 0)
    elif cpt > 1:
        scat(cpt - 1, 1)
    else:
        scat(0, 0)
    wait_scat(0, 0)
    if cpt > 1:
        wait_scat(0, 1)
    plsc.subcore_barrier()
    if esplit:
        _writeout(acc, out_hbm, s, nrec, c * nrec)
    else:
        _writeout(acc, out_hbm, s, half, c * half)


def _scatter(z, idxp, nrec, esplit):
    epad = idxp.shape[0]
    if esplit:
        cpt = epad // (NC * NS * CH)
        out_rows = NC * nrec
    else:
        cpt = epad // (NS * CH)
        out_rows = nrec
    mesh = plsc.VectorSubcoreMesh(core_axis_name="c", subcore_axis_name="s")
    kfn = pl.kernel(
        functools.partial(_scatter_body, epad, nrec, esplit),
        out_type=jax.ShapeDtypeStruct((out_rows, H), jnp.float32),
        mesh=mesh,
        scratch_types=[
            pltpu.VMEM_SHARED((ACCR, H), jnp.float32),
            pltpu.VMEM((CH, H), jnp.float32),
            pltpu.VMEM((CH, H), jnp.float32),
            pltpu.VMEM((CH,), jnp.int32),
            pltpu.VMEM((CH,), jnp.int32),
            pltpu.VMEM((2, CH), jnp.int32),
            pltpu.VMEM((ZR, H), jnp.float32),
            pltpu.SemaphoreType.DMA,
            pltpu.SemaphoreType.DMA,
            pltpu.SemaphoreType.DMA,
            pltpu.SemaphoreType.DMA,
            pltpu.SemaphoreType.DMA,
            pltpu.SemaphoreType.DMA,
        ],
    )
    return kfn(z, idxp)


def _final0_body(u_ref, sk_ref, ma_ref, mb_ref, wm_ref, w2_ref, b1_ref,
                 be_ref, o_ref):
    m = ma_ref[...] + mb_ref[...]
    t = _silu(u_ref[...]
              + jnp.dot(m, wm_ref[...], preferred_element_type=jnp.float32)
              + b1_ref[...])
    o_ref[...] = (sk_ref[...]
                  + jnp.dot(t, w2_ref[...], preferred_element_type=jnp.float32)
                  + be_ref[...])


def _final0(u, sk, m00p, wm, w2, b1, bend):
    n = u.shape[0]
    blk = 1000
    nblk = n // blk
    return pl.pallas_call(
        _final0_body,
        grid=(nblk,),
        in_specs=[
            pl.BlockSpec((blk, H), lambda i: (i, 0)),
            pl.BlockSpec((blk, H), lambda i: (i, 0)),
            pl.BlockSpec((blk, H), lambda i: (i, 0)),
            pl.BlockSpec((blk, H), lambda i: (i + nblk, 0)),
            pl.BlockSpec((H, H), lambda i: (0, 0)),
            pl.BlockSpec((H, H), lambda i: (0, 0)),
            pl.BlockSpec((1, H), lambda i: (0, 0)),
            pl.BlockSpec((1, H), lambda i: (0, 0)),
        ],
        out_specs=pl.BlockSpec((blk, H), lambda i: (i, 0)),
        out_shape=jax.ShapeDtypeStruct((n, H), jnp.float32),
    )(u, sk, m00p, m00p, wm, w2, b1.reshape(1, H), bend.reshape(1, H))


def _final1_body(u_ref, sk_ref, m01_ref, m11_ref, wa_ref, wb_ref, w2_ref,
                 b1_ref, be_ref, o_ref):
    t = (u_ref[...]
         + jnp.dot(m01_ref[...], wa_ref[...],
                   preferred_element_type=jnp.float32)
         + jnp.dot(m11_ref[...], wb_ref[...],
                   preferred_element_type=jnp.float32))
    t = _silu(t + b1_ref[...])
    o_ref[...] = (sk_ref[...]
                  + jnp.dot(t, w2_ref[...], preferred_element_type=jnp.float32)
                  + be_ref[...])


def _final1(u, sk, m01, m11, wa, wb, w2, b1, bend):
    n = u.shape[0]
    blk = 1000
    return pl.pallas_call(
        _final1_body,
        grid=(n // blk,),
        in_specs=[
            pl.BlockSpec((blk, H), lambda i: (i, 0)),
            pl.BlockSpec((blk, H), lambda i: (i, 0)),
            pl.BlockSpec((blk, H), lambda i: (i, 0)),
            pl.BlockSpec((blk, H), lambda i: (i, 0)),
            pl.BlockSpec((H, H), lambda i: (0, 0)),
            pl.BlockSpec((H, H), lambda i: (0, 0)),
            pl.BlockSpec((H, H), lambda i: (0, 0)),
            pl.BlockSpec((1, H), lambda i: (0, 0)),
            pl.BlockSpec((1, H), lambda i: (0, 0)),
        ],
        out_specs=pl.BlockSpec((blk, H), lambda i: (i, 0)),
        out_shape=jax.ShapeDtypeStruct((n, H), jnp.float32),
    )(u, sk, m01, m11, wa, wb, w2, b1.reshape(1, H), bend.reshape(1, H))


def _message(a, b, adj, inv, mw1, mb1, mw2, mb2, ew, eb, nrec, esplit):
    e = adj.shape[1]
    ninv = inv.shape[1]
    epad = _round_up(e, NC * NS * CH)
    pad = epad - e
    i0 = adj[0].astype(jnp.int32)
    i1 = adj[1].astype(jnp.int32)
    i0p = jnp.concatenate([i0, jnp.zeros((pad,), jnp.int32)])
    i1pg = jnp.concatenate([i1, jnp.zeros((pad,), jnp.int32)])
    i1ps = jnp.concatenate([i1, jnp.full((pad,), nrec, jnp.int32)])
    invp = jnp.pad(inv, ((0, pad), (0, 8 - ninv)))
    winv = jnp.pad(mw1[2 * H:], ((0, 8 - ninv), (0, 0)))
    g = _gather_sum(a, b, i0p, i1pg)
    z = _edge_mlp(g, invp, winv, mb1.reshape(1, H), mw2, mb2.reshape(1, H),
                  ew.reshape(1, H), jnp.tile(eb.reshape(1, 1), (1, H)), ninv)
    return _scatter(z, i1ps, nrec, esplit)


def kernel(x0, x1, adj_0_0, adj_0_1, adj_1_1, inv_0_0, inv_0_1, inv_1_1,
           mw1_0_0, mb1_0_0, mw2_0_0, mb2_0_0, ew_0_0, eb_0_0,
           mw1_0_1, mb1_0_1, mw2_0_1, mb2_0_1, ew_0_1, eb_0_1,
           mw1_1_1, mb1_1_1, mw2_1_1, mb2_1_1, ew_1_1, eb_1_1,
           u0w1, u0b1, u0w2, u0b2, u1w1, u1b1, u1w2, u1b2, sw, sb):
    n0 = x0.shape[0]
    n1 = x1.shape[0]
    wcat0 = jnp.concatenate(
        [mw1_0_0[:H], mw1_0_0[H:2 * H], mw1_0_1[:H], sw, u0w1[:H]], axis=1)
    wcat1 = jnp.concatenate(
        [mw1_0_1[H:2 * H], mw1_1_1[:H], mw1_1_1[H:2 * H], sw, u1w1[:H]],
        axis=1)
    a00, b00, a01, sk0, u0 = _node_proj(x0, wcat0)
    b01, a11, b11, sk1, u1 = _node_proj(x1, wcat1)

    m00p = _message(a00, b00, adj_0_0, inv_0_0, mw1_0_0, mb1_0_0, mw2_0_0,
                    mb2_0_0, ew_0_0, eb_0_0, n0, True)
    m01 = _message(a01, b01, adj_0_1, inv_0_1, mw1_0_1, mb1_0_1, mw2_0_1,
                   mb2_0_1, ew_0_1, eb_0_1, n1, False)
    m11 = _message(a11, b11, adj_1_1, inv_1_1, mw1_1_1, mb1_1_1, mw2_1_1,
                   mb2_1_1, ew_1_1, eb_1_1, n1, False)

    out0 = _final0(u0, sk0, m00p, u0w1[H:], u0w2, u0b1, u0b2 + sb)
    out1 = _final1(u1, sk1, m01, m11, u1w1[H:2 * H], u1w1[2 * H:], u1w2,
                   u1b1, u1b2 + sb)
    return (out0, out1)


# speedup vs baseline: 1.0021x; 1.0021x over previous
import functools
import jax
import jax.numpy as jnp
from jax import lax
from jax.experimental import pallas as pl
from jax.experimental.pallas import tpu as pltpu
from jax.experimental.pallas import tpu_sc as plsc

H = 128
NC = 2
NS = 16
L = 16
CH = 128
ACCR = 10240


def _silu(x):
    return x * jax.nn.sigmoid(x)


def _round_up(n, m):
    return (n + m - 1) // m * m


def _proj_body(x_ref, w_ref, *out_refs):
    y = jnp.dot(x_ref[...], w_ref[...], preferred_element_type=jnp.float32)
    for i, o in enumerate(out_refs):
        o[...] = y[:, i * H:(i + 1) * H].astype(o.dtype)


def _node_proj(x, wcat, nbf16):
    n = x.shape[0]
    nout = wcat.shape[1] // H
    blk = 1000
    dts = [jnp.bfloat16] * nbf16 + [jnp.float32] * (nout - nbf16)
    return pl.pallas_call(
        _proj_body,
        grid=(n // blk,),
        in_specs=[pl.BlockSpec((blk, H), lambda i: (i, 0)),
                  pl.BlockSpec(wcat.shape, lambda i: (0, 0))],
        out_specs=[pl.BlockSpec((blk, H), lambda i: (i, 0))
                   for _ in range(nout)],
        out_shape=[jax.ShapeDtypeStruct((n, H), dt) for dt in dts],
    )(x, wcat)


ND = 3


def _gather_body(epad, a_hbm, b_hbm, i0_hbm, i1_hbm, g_hbm,
                 i0all, i1all, av0, bv0, av1, bv1, av2, bv2,
                 sg0, sg1, sg2, ss0, ss1, ss2):
    c = lax.axis_index("c")
    s = lax.axis_index("s")
    w = s * NC + c
    cpw = epad // (NC * NS * CH)
    tb = w * cpw
    pltpu.sync_copy(i0_hbm.at[pl.ds(tb * CH, cpw * CH)], i0all)
    pltpu.sync_copy(i1_hbm.at[pl.ds(tb * CH, cpw * CH)], i1all)
    avs = (av0, av1, av2)
    bvs = (bv0, bv1, bv2)
    sgs = (sg0, sg1, sg2)
    sss = (ss0, ss1, ss2)

    def issue(j, p):
        pltpu.async_copy(a_hbm.at[i0all.at[pl.ds(j * CH, CH)]], avs[p], sgs[p])
        pltpu.async_copy(b_hbm.at[i1all.at[pl.ds(j * CH, CH)]], bvs[p], sgs[p])

    def wait_store(p):
        pltpu.make_async_copy(avs[p], g_hbm.at[0, pl.ds(0, CH)],
                              sss[p]).wait()
        pltpu.make_async_copy(bvs[p], g_hbm.at[1, pl.ds(0, CH)],
                              sss[p]).wait()

    def process(j, p):
        pltpu.make_async_copy(a_hbm.at[i0all.at[pl.ds(0, CH)]],
                              avs[p], sgs[p]).wait()
        pltpu.make_async_copy(b_hbm.at[i1all.at[pl.ds(0, CH)]],
                              bvs[p], sgs[p]).wait()
        base = (tb + j) * CH
        pltpu.async_copy(avs[p], g_hbm.at[0, pl.ds(base, CH)], sss[p])
        pltpu.async_copy(bvs[p], g_hbm.at[1, pl.ds(base, CH)], sss[p])

    def step(j, p):
        wait_store(p)
        issue(j, p)
        process(j - (ND - 1), (p - (ND - 1)) % ND)

    issue(0, 0)
    if cpw > 1:
        issue(1, 1)
    if cpw > 2:
        issue(2, 2)
        process(0, 0)
    if cpw > 3:
        wait_store(0)
        issue(3, 0)
        process(1, 1)
    ntri = (cpw - 4) // ND if cpw > 4 else 0

    def tri(t, carry):
        j = 4 + ND * t
        step(j, 4 % ND)
        step(j + 1, 5 % ND)
        step(j + 2, 6 % ND)
        return carry

    lax.fori_loop(0, ntri, tri, 0)
    for j in range(4 + ND * ntri, cpw):
        step(j, j % ND)
    if cpw > 3:
        process(cpw - 2, (cpw - 2) % ND)
        process(cpw - 1, (cpw - 1) % ND)
    elif cpw == 3:
        process(1, 1)
        process(2, 2)
    elif cpw == 2:
        process(0, 0)
        process(1, 1)
    else:
        process(0, 0)
    for p in range(min(ND, cpw)):
        wait_store(p)


def _gather_sum(a, b, i0p, i1p):
    epad = i0p.shape[0]
    cpw = epad // (NC * NS * CH)
    mesh = plsc.VectorSubcoreMesh(core_axis_name="c", subcore_axis_name="s")
    kfn = pl.kernel(
        functools.partial(_gather_body, epad),
        out_type=jax.ShapeDtypeStruct((2, epad, H), jnp.float32),
        mesh=mesh,
        scratch_types=[
            pltpu.VMEM((cpw * CH,), jnp.int32),
            pltpu.VMEM((cpw * CH,), jnp.int32),
            pltpu.VMEM((CH, H), jnp.float32),
            pltpu.VMEM((CH, H), jnp.float32),
            pltpu.VMEM((CH, H), jnp.float32),
            pltpu.VMEM((CH, H), jnp.float32),
            pltpu.VMEM((CH, H), jnp.float32),
            pltpu.VMEM((CH, H), jnp.float32),
            pltpu.SemaphoreType.DMA,
            pltpu.SemaphoreType.DMA,
            pltpu.SemaphoreType.DMA,
            pltpu.SemaphoreType.DMA,
            pltpu.SemaphoreType.DMA,
            pltpu.SemaphoreType.DMA,
        ],
    )
    return kfn(a, b, i0p, i1p)


def _edge_mlp_body(ninv, g_ref, inv_ref, winv_ref, b1_ref, w2_ref, b2_ref,
                   ew_ref, eb_ref, z_ref):
    pre = (g_ref[0].astype(jnp.float32) + g_ref[1].astype(jnp.float32)
           + b1_ref[...])
    for k in range(ninv):
        pre = pre + inv_ref[:, k:k + 1] * winv_ref[k:k + 1, :]
    m = _silu(pre)
    m2 = _silu(jnp.dot(m, w2_ref[...], preferred_element_type=jnp.float32)
               + b2_ref[...])
    logit = jnp.sum(m2 * ew_ref[...], axis=1, keepdims=True) + eb_ref[0, 0]
    z_ref[...] = m2 * jax.nn.sigmoid(logit)


def _edge_mlp(g, invp, winv, b1, w2, b2, ewr, ebr, ninv):
    epad = g.shape[1]
    blk = 512
    return pl.pallas_call(
        functools.partial(_edge_mlp_body, ninv),
        grid=(epad // blk,),
        in_specs=[
            pl.BlockSpec((2, blk, H), lambda i: (0, i, 0)),
            pl.BlockSpec((blk, 8), lambda i: (i, 0)),
            pl.BlockSpec((8, H), lambda i: (0, 0)),
            pl.BlockSpec((1, H), lambda i: (0, 0)),
            pl.BlockSpec((H, H), lambda i: (0, 0)),
            pl.BlockSpec((1, H), lambda i: (0, 0)),
            pl.BlockSpec((1, H), lambda i: (0, 0)),
            pl.BlockSpec((1, H), lambda i: (0, 0)),
        ],
        out_specs=pl.BlockSpec((blk, H), lambda i: (i, 0)),
        out_shape=jax.ShapeDtypeStruct((epad, H), jnp.float32),
    )(g, invp, winv, b1, w2, b2, ewr, ebr)


ZR = 32


def _zero_fill(zrow):
    def zrowfill(r, carry):
        for k in range(H // L):
            zrow[r, pl.ds(k * L, L)] = jnp.zeros((L,), jnp.float32)
        return carry

    lax.fori_loop(0, ZR, zrowfill, 0)


def _zero_acc(acc, zrow, s):
    zpt = ACCR // NS

    def zchunk(q, carry):
        pltpu.sync_copy(zrow, acc.at[pl.ds(s * zpt + q * ZR, ZR)])
        return carry

    lax.fori_loop(0, zpt // ZR, zchunk, 0)


def _writeout(acc, out_hbm, s, nrows, out_base):
    nch = nrows // 8
    npt = (nch + NS - 1) // NS

    def wo(j, carry):
        ch = j * NS + s

        @pl.when(ch < nch)
        def _():
            pltpu.sync_copy(acc.at[pl.ds(ch * 8, 8)],
                            out_hbm.at[pl.ds(out_base + ch * 8, 8)])
        return carry

    lax.fori_loop(0, npt, wo, 0)


def _scatter_body(epad, nrec, esplit, z_hbm, idx_hbm, out_hbm,
                  acc, zv0, zv1, iv0, iv1, lbuf, zrow,
                  sl0, sl1, si0, si1, sa0, sa1):
    c = lax.axis_index("c")
    s = lax.axis_index("s")
    if esplit:
        half = nrec
        cpt = epad // (NC * NS * CH)
        tb = (c * NS + s) * cpt
        lo = c * 0
    else:
        half = nrec // 2
        cpt = epad // (NS * CH)
        tb = s * cpt
        lo = c * half
    _zero_fill(zrow)
    _zero_acc(acc, zrow, s)
    plsc.subcore_barrier()

    zvs = (zv0, zv1)
    ivs = (iv0, iv1)
    sls = (sl0, sl1)
    sis = (si0, si1)
    sas = (sa0, sa1)

    def zload(j, p):
        pltpu.async_copy(z_hbm.at[pl.ds((tb + j) * CH, CH)], zvs[p], sls[p])
        pltpu.async_copy(idx_hbm.at[pl.ds((tb + j) * CH, CH)], ivs[p], sis[p])

    def scat(j, p):
        pltpu.make_async_copy(idx_hbm.at[pl.ds(0, CH)], ivs[p],
                              sis[p]).wait()
        for k in range(CH // L):
            sl = pl.ds(k * L, L)
            t = ivs[p][sl] - lo
            ok = (t >= 0) & (t < half)
            lbuf[p, sl] = jnp.where(ok, t, half)
        pltpu.make_async_copy(z_hbm.at[pl.ds(0, CH)], zvs[p], sls[p]).wait()
        pltpu.async_copy(zvs[p], acc.at[lbuf.at[p]], sas[p], add=True)

    def wait_scat(j, p):
        pltpu.make_async_copy(zvs[p], acc.at[lbuf.at[p]], sas[p]).wait()

    zload(0, 0)
    if cpt > 1:
        zload(1, 1)
        scat(0, 0)
    npairs = (cpt - 2) // 2 if cpt > 2 else 0

    def pair(t, carry):
        j = 2 + 2 * t
        wait_scat(j - 2, 0)
        zload(j, 0)
        scat(j - 1, 1)
        wait_scat(j - 1, 1)
        zload(j + 1, 1)
        scat(j, 0)
        return carry

    lax.fori_loop(0, npairs, pair, 0)
    rem = 2 + 2 * npairs
    if cpt > 2 and rem < cpt:
        wait_scat(rem - 2, 0)
        zload(rem, 0)
        scat(rem - 1, 1)
        scat(rem, 0)
    elif cpt > 1:
        scat(cpt - 1, 1)
    else:
        scat(0, 0)
    wait_scat(0, 0)
    if cpt > 1:
        wait_scat(0, 1)
    plsc.subcore_barrier()
    if esplit:
        _writeout(acc, out_hbm, s, nrec, c * nrec)
    else:
        _writeout(acc, out_hbm, s, half, c * half)


def _scatter(z, idxp, nrec, esplit):
    epad = idxp.shape[0]
    if esplit:
        cpt = epad // (NC * NS * CH)
        out_rows = NC * nrec
    else:
        cpt = epad // (NS * CH)
        out_rows = nrec
    mesh = plsc.VectorSubcoreMesh(core_axis_name="c", subcore_axis_name="s")
    kfn = pl.kernel(
        functools.partial(_scatter_body, epad, nrec, esplit),
        out_type=jax.ShapeDtypeStruct((out_rows, H), jnp.float32),
        mesh=mesh,
        scratch_types=[
            pltpu.VMEM_SHARED((ACCR, H), jnp.float32),
            pltpu.VMEM((CH, H), jnp.float32),
            pltpu.VMEM((CH, H), jnp.float32),
            pltpu.VMEM((CH,), jnp.int32),
            pltpu.VMEM((CH,), jnp.int32),
            pltpu.VMEM((2, CH), jnp.int32),
            pltpu.VMEM((ZR, H), jnp.float32),
            pltpu.SemaphoreType.DMA,
            pltpu.SemaphoreType.DMA,
            pltpu.SemaphoreType.DMA,
            pltpu.SemaphoreType.DMA,
            pltpu.SemaphoreType.DMA,
            pltpu.SemaphoreType.DMA,
        ],
    )
    return kfn(z, idxp)


def _final0_body(u_ref, sk_ref, ma_ref, mb_ref, wm_ref, w2_ref, b1_ref,
                 be_ref, o_ref):
    m = ma_ref[...] + mb_ref[...]
    t = _silu(u_ref[...]
              + jnp.dot(m, wm_ref[...], preferred_element_type=jnp.float32)
              + b1_ref[...])
    o_ref[...] = (sk_ref[...]
                  + jnp.dot(t, w2_ref[...], preferred_element_type=jnp.float32)
                  + be_ref[...])


def _final0(u, sk, m00p, wm, w2, b1, bend):
    n = u.shape[0]
    blk = 1000
    nblk = n // blk
    return pl.pallas_call(
        _final0_body,
        grid=(nblk,),
        in_specs=[
            pl.BlockSpec((blk, H), lambda i: (i, 0)),
            pl.BlockSpec((blk, H), lambda i: (i, 0)),
            pl.BlockSpec((blk, H), lambda i: (i, 0)),
            pl.BlockSpec((blk, H), lambda i: (i + nblk, 0)),
            pl.BlockSpec((H, H), lambda i: (0, 0)),
            pl.BlockSpec((H, H), lambda i: (0, 0)),
            pl.BlockSpec((1, H), lambda i: (0, 0)),
            pl.BlockSpec((1, H), lambda i: (0, 0)),
        ],
        out_specs=pl.BlockSpec((blk, H), lambda i: (i, 0)),
        out_shape=jax.ShapeDtypeStruct((n, H), jnp.float32),
    )(u, sk, m00p, m00p, wm, w2, b1.reshape(1, H), bend.reshape(1, H))


def _final1_body(u_ref, sk_ref, m01_ref, m11_ref, wa_ref, wb_ref, w2_ref,
                 b1_ref, be_ref, o_ref):
    t = (u_ref[...]
         + jnp.dot(m01_ref[...], wa_ref[...],
                   preferred_element_type=jnp.float32)
         + jnp.dot(m11_ref[...], wb_ref[...],
                   preferred_element_type=jnp.float32))
    t = _silu(t + b1_ref[...])
    o_ref[...] = (sk_ref[...]
                  + jnp.dot(t, w2_ref[...], preferred_element_type=jnp.float32)
                  + be_ref[...])


def _final1(u, sk, m01, m11, wa, wb, w2, b1, bend):
    n = u.shape[0]
    blk = 1000
    return pl.pallas_call(
        _final1_body,
        grid=(n // blk,),
        in_specs=[
            pl.BlockSpec((blk, H), lambda i: (i, 0)),
            pl.BlockSpec((blk, H), lambda i: (i, 0)),
            pl.BlockSpec((blk, H), lambda i: (i, 0)),
            pl.BlockSpec((blk, H), lambda i: (i, 0)),
            pl.BlockSpec((H, H), lambda i: (0, 0)),
            pl.BlockSpec((H, H), lambda i: (0, 0)),
            pl.BlockSpec((H, H), lambda i: (0, 0)),
            pl.BlockSpec((1, H), lambda i: (0, 0)),
            pl.BlockSpec((1, H), lambda i: (0, 0)),
        ],
        out_specs=pl.BlockSpec((blk, H), lambda i: (i, 0)),
        out_shape=jax.ShapeDtypeStruct((n, H), jnp.float32),
    )(u, sk, m01, m11, wa, wb, w2, b1.reshape(1, H), bend.reshape(1, H))


def _message(a, b, adj, inv, mw1, mb1, mw2, mb2, ew, eb, nrec, esplit):
    e = adj.shape[1]
    ninv = inv.shape[1]
    epad = _round_up(e, NC * NS * CH)
    pad = epad - e
    i0 = adj[0].astype(jnp.int32)
    i1 = adj[1].astype(jnp.int32)
    i0p = jnp.concatenate([i0, jnp.zeros((pad,), jnp.int32)])
    i1pg = jnp.concatenate([i1, jnp.zeros((pad,), jnp.int32)])
    i1ps = jnp.concatenate([i1, jnp.full((pad,), nrec, jnp.int32)])
    invp = jnp.pad(inv, ((0, pad), (0, 8 - ninv)))
    winv = jnp.pad(mw1[2 * H:], ((0, 8 - ninv), (0, 0)))
    g = _gather_sum(a, b, i0p, i1pg)
    z = _edge_mlp(g, invp, winv, mb1.reshape(1, H), mw2, mb2.reshape(1, H),
                  ew.reshape(1, H), jnp.tile(eb.reshape(1, 1), (1, H)), ninv)
    return _scatter(z, i1ps, nrec, esplit)


def kernel(x0, x1, adj_0_0, adj_0_1, adj_1_1, inv_0_0, inv_0_1, inv_1_1,
           mw1_0_0, mb1_0_0, mw2_0_0, mb2_0_0, ew_0_0, eb_0_0,
           mw1_0_1, mb1_0_1, mw2_0_1, mb2_0_1, ew_0_1, eb_0_1,
           mw1_1_1, mb1_1_1, mw2_1_1, mb2_1_1, ew_1_1, eb_1_1,
           u0w1, u0b1, u0w2, u0b2, u1w1, u1b1, u1w2, u1b2, sw, sb):
    n0 = x0.shape[0]
    n1 = x1.shape[0]
    wcat0 = jnp.concatenate(
        [mw1_0_0[:H], mw1_0_0[H:2 * H], mw1_0_1[:H], sw, u0w1[:H]], axis=1)
    wcat1 = jnp.concatenate(
        [mw1_0_1[H:2 * H], mw1_1_1[:H], mw1_1_1[H:2 * H], sw, u1w1[:H]],
        axis=1)
    a00, b00, a01, sk0, u0 = _node_proj(x0, wcat0, 0)
    b01, a11, b11, sk1, u1 = _node_proj(x1, wcat1, 0)

    m00p = _message(a00, b00, adj_0_0, inv_0_0, mw1_0_0, mb1_0_0, mw2_0_0,
                    mb2_0_0, ew_0_0, eb_0_0, n0, True)
    m01 = _message(a01, b01, adj_0_1, inv_0_1, mw1_0_1, mb1_0_1, mw2_0_1,
                   mb2_0_1, ew_0_1, eb_0_1, n1, False)
    m11 = _message(a11, b11, adj_1_1, inv_1_1, mw1_1_1, mb1_1_1, mw2_1_1,
                   mb2_1_1, ew_1_1, eb_1_1, n1, False)

    out0 = _final0(u0, sk0, m00p, u0w1[H:], u0w2, u0b1, u0b2 + sb)
    out1 = _final1(u1, sk1, m01, m11, u1w1[H:2 * H], u1w1[2 * H:], u1w2,
                   u1b1, u1b2 + sb)
    return (out0, out1)


# speedup vs baseline: 1.0496x; 1.0474x over previous
import functools
import jax
import jax.numpy as jnp
from jax import lax
from jax.experimental import pallas as pl
from jax.experimental.pallas import tpu as pltpu
from jax.experimental.pallas import tpu_sc as plsc

H = 128
NC = 2
NS = 16
L = 16
CH = 128
ACCR = 10240


def _silu(x):
    return x * jax.nn.sigmoid(x)


def _round_up(n, m):
    return (n + m - 1) // m * m


def _proj_body(x_ref, w_ref, *out_refs):
    y = jnp.dot(x_ref[...], w_ref[...], preferred_element_type=jnp.float32)
    for i, o in enumerate(out_refs):
        o[...] = y[:, i * H:(i + 1) * H].astype(o.dtype)


def _node_proj(x, wcat, nbf16):
    n = x.shape[0]
    nout = wcat.shape[1] // H
    blk = 1000
    dts = [jnp.bfloat16] * nbf16 + [jnp.float32] * (nout - nbf16)
    return pl.pallas_call(
        _proj_body,
        grid=(n // blk,),
        in_specs=[pl.BlockSpec((blk, H), lambda i: (i, 0)),
                  pl.BlockSpec(wcat.shape, lambda i: (0, 0))],
        out_specs=[pl.BlockSpec((blk, H), lambda i: (i, 0))
                   for _ in range(nout)],
        out_shape=[jax.ShapeDtypeStruct((n, H), dt) for dt in dts],
    )(x, wcat)


ND = 3


def _gather_body(epad, a_hbm, b_hbm, i0_hbm, i1_hbm, g_hbm,
                 i0all, i1all, av0, bv0, av1, bv1, av2, bv2,
                 sg0, sg1, sg2, ss0, ss1, ss2):
    c = lax.axis_index("c")
    s = lax.axis_index("s")
    w = s * NC + c
    cpw = epad // (NC * NS * CH)
    tb = w * cpw
    pltpu.sync_copy(i0_hbm.at[pl.ds(tb * CH, cpw * CH)], i0all)
    pltpu.sync_copy(i1_hbm.at[pl.ds(tb * CH, cpw * CH)], i1all)
    avs = (av0, av1, av2)
    bvs = (bv0, bv1, bv2)
    sgs = (sg0, sg1, sg2)
    sss = (ss0, ss1, ss2)

    def issue(j, p):
        pltpu.async_copy(a_hbm.at[i0all.at[pl.ds(j * CH, CH)]], avs[p], sgs[p])
        pltpu.async_copy(b_hbm.at[i1all.at[pl.ds(j * CH, CH)]], bvs[p], sgs[p])

    def wait_store(p):
        pltpu.make_async_copy(avs[p], g_hbm.at[pl.ds(0, CH)], sss[p]).wait()

    def process(j, p):
        pltpu.make_async_copy(a_hbm.at[i0all.at[pl.ds(0, CH)]],
                              avs[p], sgs[p]).wait()
        pltpu.make_async_copy(b_hbm.at[i1all.at[pl.ds(0, CH)]],
                              bvs[p], sgs[p]).wait()
        av, bv = avs[p], bvs[p]

        def row(r, carry2):
            for k in range(H // L):
                sl = pl.ds(k * L, L)
                av[r, sl] = av[r, sl] + bv[r, sl]
            return carry2

        lax.fori_loop(0, CH, row, 0, unroll=4)
        pltpu.async_copy(av, g_hbm.at[pl.ds((tb + j) * CH, CH)], sss[p])

    def step(j, p):
        wait_store(p)
        issue(j, p)
        process(j - (ND - 1), (p - (ND - 1)) % ND)

    issue(0, 0)
    if cpw > 1:
        issue(1, 1)
    if cpw > 2:
        issue(2, 2)
        process(0, 0)
    if cpw > 3:
        wait_store(0)
        issue(3, 0)
        process(1, 1)
    ntri = (cpw - 4) // ND if cpw > 4 else 0

    def tri(t, carry):
        j = 4 + ND * t
        step(j, 4 % ND)
        step(j + 1, 5 % ND)
        step(j + 2, 6 % ND)
        return carry

    lax.fori_loop(0, ntri, tri, 0)
    for j in range(4 + ND * ntri, cpw):
        step(j, j % ND)
    if cpw > 3:
        process(cpw - 2, (cpw - 2) % ND)
        process(cpw - 1, (cpw - 1) % ND)
    elif cpw == 3:
        process(1, 1)
        process(2, 2)
    elif cpw == 2:
        process(0, 0)
        process(1, 1)
    else:
        process(0, 0)
    for p in range(min(ND, cpw)):
        wait_store(p)


def _gather_sum(a, b, i0p, i1p):
    epad = i0p.shape[0]
    cpw = epad // (NC * NS * CH)
    mesh = plsc.VectorSubcoreMesh(core_axis_name="c", subcore_axis_name="s")
    kfn = pl.kernel(
        functools.partial(_gather_body, epad),
        out_type=jax.ShapeDtypeStruct((epad, H), jnp.float32),
        mesh=mesh,
        scratch_types=[
            pltpu.VMEM((cpw * CH,), jnp.int32),
            pltpu.VMEM((cpw * CH,), jnp.int32),
            pltpu.VMEM((CH, H), jnp.float32),
            pltpu.VMEM((CH, H), jnp.float32),
            pltpu.VMEM((CH, H), jnp.float32),
            pltpu.VMEM((CH, H), jnp.float32),
            pltpu.VMEM((CH, H), jnp.float32),
            pltpu.VMEM((CH, H), jnp.float32),
            pltpu.SemaphoreType.DMA,
            pltpu.SemaphoreType.DMA,
            pltpu.SemaphoreType.DMA,
            pltpu.SemaphoreType.DMA,
            pltpu.SemaphoreType.DMA,
            pltpu.SemaphoreType.DMA,
        ],
    )
    return kfn(a, b, i0p, i1p)


def _edge_mlp_body(ninv, g_ref, inv_ref, winv_ref, b1_ref, w2_ref, b2_ref,
                   ew_ref, eb_ref, z_ref):
    pre = g_ref[...] + b1_ref[...]
    for k in range(ninv):
        pre = pre + inv_ref[:, k:k + 1] * winv_ref[k:k + 1, :]
    m = _silu(pre)
    m2 = _silu(jnp.dot(m, w2_ref[...], preferred_element_type=jnp.float32)
               + b2_ref[...])
    logit = jnp.sum(m2 * ew_ref[...], axis=1, keepdims=True) + eb_ref[0, 0]
    z_ref[...] = m2 * jax.nn.sigmoid(logit)


def _edge_mlp(g, invp, winv, b1, w2, b2, ewr, ebr, ninv):
    epad = g.shape[0]
    blk = 512
    return pl.pallas_call(
        functools.partial(_edge_mlp_body, ninv),
        grid=(epad // blk,),
        in_specs=[
            pl.BlockSpec((blk, H), lambda i: (i, 0)),
            pl.BlockSpec((blk, 8), lambda i: (i, 0)),
            pl.BlockSpec((8, H), lambda i: (0, 0)),
            pl.BlockSpec((1, H), lambda i: (0, 0)),
            pl.BlockSpec((H, H), lambda i: (0, 0)),
            pl.BlockSpec((1, H), lambda i: (0, 0)),
            pl.BlockSpec((1, H), lambda i: (0, 0)),
            pl.BlockSpec((1, H), lambda i: (0, 0)),
        ],
        out_specs=pl.BlockSpec((blk, H), lambda i: (i, 0)),
        out_shape=jax.ShapeDtypeStruct((epad, H), jnp.float32),
    )(g, invp, winv, b1, w2, b2, ewr, ebr)


ZR = 32


def _zero_fill(zrow):
    def zrowfill(r, carry):
        for k in range(H // L):
            zrow[r, pl.ds(k * L, L)] = jnp.zeros((L,), jnp.float32)
        return carry

    lax.fori_loop(0, ZR, zrowfill, 0)


def _zero_acc(acc, zrow, s):
    zpt = ACCR // NS

    def zchunk(q, carry):
        pltpu.sync_copy(zrow, acc.at[pl.ds(s * zpt + q * ZR, ZR)])
        return carry

    lax.fori_loop(0, zpt // ZR, zchunk, 0)


def _writeout(acc, out_hbm, s, nrows, out_base):
    nch = nrows // 8
    npt = (nch + NS - 1) // NS

    def wo(j, carry):
        ch = j * NS + s

        @pl.when(ch < nch)
        def _():
            pltpu.sync_copy(acc.at[pl.ds(ch * 8, 8)],
                            out_hbm.at[pl.ds(out_base + ch * 8, 8)])
        return carry

    lax.fori_loop(0, npt, wo, 0)


def _scatter_body(epad, nrec, esplit, z_hbm, idx_hbm, out_hbm,
                  acc, zv0, zv1, iv0, iv1, lbuf, zrow,
                  sl0, sl1, si0, si1, sa0, sa1):
    c = lax.axis_index("c")
    s = lax.axis_index("s")
    if esplit:
        half = nrec
        cpt = epad // (NC * NS * CH)
        tb = (c * NS + s) * cpt
        lo = c * 0
    else:
        half = nrec // 2
        cpt = epad // (NS * CH)
        tb = s * cpt
        lo = c * half
    _zero_fill(zrow)
    _zero_acc(acc, zrow, s)
    plsc.subcore_barrier()

    zvs = (zv0, zv1)
    ivs = (iv0, iv1)
    sls = (sl0, sl1)
    sis = (si0, si1)
    sas = (sa0, sa1)

    def zload(j, p):
        pltpu.async_copy(z_hbm.at[pl.ds((tb + j) * CH, CH)], zvs[p], sls[p])
        pltpu.async_copy(idx_hbm.at[pl.ds((tb + j) * CH, CH)], ivs[p], sis[p])

    def scat(j, p):
        pltpu.make_async_copy(idx_hbm.at[pl.ds(0, CH)], ivs[p],
                              sis[p]).wait()
        for k in range(CH // L):
            sl = pl.ds(k * L, L)
            t = ivs[p][sl] - lo
            ok = (t >= 0) & (t < half)
            lbuf[p, sl] = jnp.where(ok, t, half)
        pltpu.make_async_copy(z_hbm.at[pl.ds(0, CH)], zvs[p], sls[p]).wait()
        pltpu.async_copy(zvs[p], acc.at[lbuf.at[p]], sas[p], add=True)

    def wait_scat(j, p):
        pltpu.make_async_copy(zvs[p], acc.at[lbuf.at[p]], sas[p]).wait()

    zload(0, 0)
    if cpt > 1:
        zload(1, 1)
        scat(0, 0)
    npairs = (cpt - 2) // 2 if cpt > 2 else 0

    def pair(t, carry):
        j = 2 + 2 * t
        wait_scat(j - 2, 0)
        zload(j, 0)
        scat(j - 1, 1)
        wait_scat(j - 1, 1)
        zload(j + 1, 1)
        scat(j, 0)
        return carry

    lax.fori_loop(0, npairs, pair, 0)
    rem = 2 + 2 * npairs
    if cpt > 2 and rem < cpt:
        wait_scat(rem - 2, 0)
        zload(rem, 0)
        scat(rem - 1, 1)
        scat(rem, 0)
    elif cpt > 1:
        scat(cpt - 1, 1)
    else:
        scat(0, 0)
    wait_scat(0, 0)
    if cpt > 1:
        wait_scat(0, 1)
    plsc.subcore_barrier()
    if esplit:
        _writeout(acc, out_hbm, s, nrec, c * nrec)
    else:
        _writeout(acc, out_hbm, s, half, c * half)


def _scatter(z, idxp, nrec, esplit):
    epad = idxp.shape[0]
    if esplit:
        cpt = epad // (NC * NS * CH)
        out_rows = NC * nrec
    else:
        cpt = epad // (NS * CH)
        out_rows = nrec
    mesh = plsc.VectorSubcoreMesh(core_axis_name="c", subcore_axis_name="s")
    kfn = pl.kernel(
        functools.partial(_scatter_body, epad, nrec, esplit),
        out_type=jax.ShapeDtypeStruct((out_rows, H), jnp.float32),
        mesh=mesh,
        scratch_types=[
            pltpu.VMEM_SHARED((ACCR, H), jnp.float32),
            pltpu.VMEM((CH, H), jnp.float32),
            pltpu.VMEM((CH, H), jnp.float32),
            pltpu.VMEM((CH,), jnp.int32),
            pltpu.VMEM((CH,), jnp.int32),
            pltpu.VMEM((2, CH), jnp.int32),
            pltpu.VMEM((ZR, H), jnp.float32),
            pltpu.SemaphoreType.DMA,
            pltpu.SemaphoreType.DMA,
            pltpu.SemaphoreType.DMA,
            pltpu.SemaphoreType.DMA,
            pltpu.SemaphoreType.DMA,
            pltpu.SemaphoreType.DMA,
        ],
    )
    return kfn(z, idxp)


def _final0_body(u_ref, sk_ref, ma_ref, mb_ref, wm_ref, w2_ref, b1_ref,
                 be_ref, o_ref):
    m = ma_ref[...] + mb_ref[...]
    t = _silu(u_ref[...]
              + jnp.dot(m, wm_ref[...], preferred_element_type=jnp.float32)
              + b1_ref[...])
    o_ref[...] = (sk_ref[...]
                  + jnp.dot(t, w2_ref[...], preferred_element_type=jnp.float32)
                  + be_ref[...])


def _final0(u, sk, m00p, wm, w2, b1, bend):
    n = u.shape[0]
    blk = 1000
    nblk = n // blk
    return pl.pallas_call(
        _final0_body,
        grid=(nblk,),
        in_specs=[
            pl.BlockSpec((blk, H), lambda i: (i, 0)),
            pl.BlockSpec((blk, H), lambda i: (i, 0)),
            pl.BlockSpec((blk, H), lambda i: (i, 0)),
            pl.BlockSpec((blk, H), lambda i: (i + nblk, 0)),
            pl.BlockSpec((H, H), lambda i: (0, 0)),
            pl.BlockSpec((H, H), lambda i: (0, 0)),
            pl.BlockSpec((1, H), lambda i: (0, 0)),
            pl.BlockSpec((1, H), lambda i: (0, 0)),
        ],
        out_specs=pl.BlockSpec((blk, H), lambda i: (i, 0)),
        out_shape=jax.ShapeDtypeStruct((n, H), jnp.float32),
    )(u, sk, m00p, m00p, wm, w2, b1.reshape(1, H), bend.reshape(1, H))


def _final1_body(u_ref, sk_ref, m01_ref, m11_ref, wa_ref, wb_ref, w2_ref,
                 b1_ref, be_ref, o_ref):
    t = (u_ref[...]
         + jnp.dot(m01_ref[...], wa_ref[...],
                   preferred_element_type=jnp.float32)
         + jnp.dot(m11_ref[...], wb_ref[...],
                   preferred_element_type=jnp.float32))
    t = _silu(t + b1_ref[...])
    o_ref[...] = (sk_ref[...]
                  + jnp.dot(t, w2_ref[...], preferred_element_type=jnp.float32)
                  + be_ref[...])


def _final1(u, sk, m01, m11, wa, wb, w2, b1, bend):
    n = u.shape[0]
    blk = 1000
    return pl.pallas_call(
        _final1_body,
        grid=(n // blk,),
        in_specs=[
            pl.BlockSpec((blk, H), lambda i: (i, 0)),
            pl.BlockSpec((blk, H), lambda i: (i, 0)),
            pl.BlockSpec((blk, H), lambda i: (i, 0)),
            pl.BlockSpec((blk, H), lambda i: (i, 0)),
            pl.BlockSpec((H, H), lambda i: (0, 0)),
            pl.BlockSpec((H, H), lambda i: (0, 0)),
            pl.BlockSpec((H, H), lambda i: (0, 0)),
            pl.BlockSpec((1, H), lambda i: (0, 0)),
            pl.BlockSpec((1, H), lambda i: (0, 0)),
        ],
        out_specs=pl.BlockSpec((blk, H), lambda i: (i, 0)),
        out_shape=jax.ShapeDtypeStruct((n, H), jnp.float32),
    )(u, sk, m01, m11, wa, wb, w2, b1.reshape(1, H), bend.reshape(1, H))


def _message(a, b, adj, inv, mw1, mb1, mw2, mb2, ew, eb, nrec, esplit):
    e = adj.shape[1]
    ninv = inv.shape[1]
    epad = _round_up(e, NC * NS * CH)
    pad = epad - e
    i0 = adj[0].astype(jnp.int32)
    i1 = adj[1].astype(jnp.int32)
    i0p = jnp.concatenate([i0, jnp.zeros((pad,), jnp.int32)])
    i1pg = jnp.concatenate([i1, jnp.zeros((pad,), jnp.int32)])
    i1ps = jnp.concatenate([i1, jnp.full((pad,), nrec, jnp.int32)])
    invp = jnp.pad(inv, ((0, pad), (0, 8 - ninv)))
    winv = jnp.pad(mw1[2 * H:], ((0, 8 - ninv), (0, 0)))
    g = _gather_sum(a, b, i0p, i1pg)
    z = _edge_mlp(g, invp, winv, mb1.reshape(1, H), mw2, mb2.reshape(1, H),
                  ew.reshape(1, H), jnp.tile(eb.reshape(1, 1), (1, H)), ninv)
    return _scatter(z, i1ps, nrec, esplit)


def kernel(x0, x1, adj_0_0, adj_0_1, adj_1_1, inv_0_0, inv_0_1, inv_1_1,
           mw1_0_0, mb1_0_0, mw2_0_0, mb2_0_0, ew_0_0, eb_0_0,
           mw1_0_1, mb1_0_1, mw2_0_1, mb2_0_1, ew_0_1, eb_0_1,
           mw1_1_1, mb1_1_1, mw2_1_1, mb2_1_1, ew_1_1, eb_1_1,
           u0w1, u0b1, u0w2, u0b2, u1w1, u1b1, u1w2, u1b2, sw, sb):
    n0 = x0.shape[0]
    n1 = x1.shape[0]
    wcat0 = jnp.concatenate(
        [mw1_0_0[:H], mw1_0_0[H:2 * H], mw1_0_1[:H], sw, u0w1[:H]], axis=1)
    wcat1 = jnp.concatenate(
        [mw1_0_1[H:2 * H], mw1_1_1[:H], mw1_1_1[H:2 * H], sw, u1w1[:H]],
        axis=1)
    a00, b00, a01, sk0, u0 = _node_proj(x0, wcat0, 0)
    b01, a11, b11, sk1, u1 = _node_proj(x1, wcat1, 0)

    m00p = _message(a00, b00, adj_0_0, inv_0_0, mw1_0_0, mb1_0_0, mw2_0_0,
                    mb2_0_0, ew_0_0, eb_0_0, n0, True)
    m01 = _message(a01, b01, adj_0_1, inv_0_1, mw1_0_1, mb1_0_1, mw2_0_1,
                   mb2_0_1, ew_0_1, eb_0_1, n1, False)
    m11 = _message(a11, b11, adj_1_1, inv_1_1, mw1_1_1, mb1_1_1, mw2_1_1,
                   mb2_1_1, ew_1_1, eb_1_1, n1, False)

    out0 = _final0(u0, sk0, m00p, u0w1[H:], u0w2, u0b1, u0b2 + sb)
    out1 = _final1(u1, sk1, m01, m11, u1w1[H:2 * H], u1w1[2 * H:], u1w2,
                   u1b1, u1b2 + sb)
    return (out0, out1)


# speedup vs baseline: 1.2022x; 1.1455x over previous
import functools
import jax
import jax.numpy as jnp
from jax import lax
from jax.experimental import pallas as pl
from jax.experimental.pallas import tpu as pltpu
from jax.experimental.pallas import tpu_sc as plsc

H = 128
NC = 2
NS = 16
L = 16
CH = 128
ACCR = 10240


def _silu(x):
    return x * jax.nn.sigmoid(x)


def _round_up(n, m):
    return (n + m - 1) // m * m


def _proj_body(x_ref, w_ref, *out_refs):
    y = jnp.dot(x_ref[...], w_ref[...], preferred_element_type=jnp.float32)
    for i, o in enumerate(out_refs):
        o[...] = y[:, i * H:(i + 1) * H].astype(o.dtype)


def _node_proj(x, wcat, nbf16):
    n = x.shape[0]
    nout = wcat.shape[1] // H
    blk = 1000
    dts = [jnp.bfloat16] * nbf16 + [jnp.float32] * (nout - nbf16)
    return pl.pallas_call(
        _proj_body,
        grid=(n // blk,),
        in_specs=[pl.BlockSpec((blk, H), lambda i: (i, 0)),
                  pl.BlockSpec(wcat.shape, lambda i: (0, 0))],
        out_specs=[pl.BlockSpec((blk, H), lambda i: (i, 0))
                   for _ in range(nout)],
        out_shape=[jax.ShapeDtypeStruct((n, H), dt) for dt in dts],
    )(x, wcat)


ND = 3


def _gather_body(epad, a_hbm, b_hbm, i0_hbm, i1_hbm, g_hbm,
                 i0all, i1all, av0, bv0, av1, bv1, av2, bv2,
                 sg0, sg1, sg2, ss0, ss1, ss2):
    c = lax.axis_index("c")
    s = lax.axis_index("s")
    w = s * NC + c
    cpw = epad // (NC * NS * CH)
    tb = w * cpw
    pltpu.sync_copy(i0_hbm.at[pl.ds(tb * CH, cpw * CH)], i0all)
    pltpu.sync_copy(i1_hbm.at[pl.ds(tb * CH, cpw * CH)], i1all)
    avs = (av0, av1, av2)
    bvs = (bv0, bv1, bv2)
    sgs = (sg0, sg1, sg2)
    sss = (ss0, ss1, ss2)

    def issue(j, p):
        pltpu.async_copy(a_hbm.at[i0all.at[pl.ds(j * CH, CH)]], avs[p], sgs[p])
        pltpu.async_copy(b_hbm.at[i1all.at[pl.ds(j * CH, CH)]], bvs[p], sgs[p])

    def wait_store(p):
        pltpu.make_async_copy(avs[p], g_hbm.at[pl.ds(0, CH)], sss[p]).wait()

    def process(j, p):
        pltpu.make_async_copy(a_hbm.at[i0all.at[pl.ds(0, CH)]],
                              avs[p], sgs[p]).wait()
        pltpu.make_async_copy(b_hbm.at[i1all.at[pl.ds(0, CH)]],
                              bvs[p], sgs[p]).wait()
        av, bv = avs[p], bvs[p]

        def row(r, carry2):
            for k in range(H // L):
                sl = pl.ds(k * L, L)
                av[r, sl] = av[r, sl] + bv[r, sl]
            return carry2

        lax.fori_loop(0, CH, row, 0, unroll=8)
        pltpu.async_copy(av, g_hbm.at[pl.ds((tb + j) * CH, CH)], sss[p])

    def step(j, p):
        wait_store(p)
        issue(j, p)
        process(j - (ND - 1), (p - (ND - 1)) % ND)

    issue(0, 0)
    if cpw > 1:
        issue(1, 1)
    if cpw > 2:
        issue(2, 2)
        process(0, 0)
    if cpw > 3:
        wait_store(0)
        issue(3, 0)
        process(1, 1)
    ntri = (cpw - 4) // ND if cpw > 4 else 0

    def tri(t, carry):
        j = 4 + ND * t
        step(j, 4 % ND)
        step(j + 1, 5 % ND)
        step(j + 2, 6 % ND)
        return carry

    lax.fori_loop(0, ntri, tri, 0)
    for j in range(4 + ND * ntri, cpw):
        step(j, j % ND)
    if cpw > 3:
        process(cpw - 2, (cpw - 2) % ND)
        process(cpw - 1, (cpw - 1) % ND)
    elif cpw == 3:
        process(1, 1)
        process(2, 2)
    elif cpw == 2:
        process(0, 0)
        process(1, 1)
    else:
        process(0, 0)
    for p in range(min(ND, cpw)):
        wait_store(p)


def _gather_sum(a, b, i0p, i1p):
    epad = i0p.shape[0]
    cpw = epad // (NC * NS * CH)
    mesh = plsc.VectorSubcoreMesh(core_axis_name="c", subcore_axis_name="s")
    kfn = pl.kernel(
        functools.partial(_gather_body, epad),
        out_type=jax.ShapeDtypeStruct((epad, H), jnp.float32),
        mesh=mesh,
        scratch_types=[
            pltpu.VMEM((cpw * CH,), jnp.int32),
            pltpu.VMEM((cpw * CH,), jnp.int32),
            pltpu.VMEM((CH, H), jnp.float32),
            pltpu.VMEM((CH, H), jnp.float32),
            pltpu.VMEM((CH, H), jnp.float32),
            pltpu.VMEM((CH, H), jnp.float32),
            pltpu.VMEM((CH, H), jnp.float32),
            pltpu.VMEM((CH, H), jnp.float32),
            pltpu.SemaphoreType.DMA,
            pltpu.SemaphoreType.DMA,
            pltpu.SemaphoreType.DMA,
            pltpu.SemaphoreType.DMA,
            pltpu.SemaphoreType.DMA,
            pltpu.SemaphoreType.DMA,
        ],
    )
    return kfn(a, b, i0p, i1p)


def _edge_mlp_body(ninv, g_ref, inv_ref, winv_ref, b1_ref, w2_ref, b2_ref,
                   ew_ref, eb_ref, z_ref):
    pre = g_ref[...] + b1_ref[...]
    for k in range(ninv):
        pre = pre + inv_ref[:, k:k + 1] * winv_ref[k:k + 1, :]
    m = _silu(pre)
    m2 = _silu(jnp.dot(m, w2_ref[...], preferred_element_type=jnp.float32)
               + b2_ref[...])
    logit = jnp.sum(m2 * ew_ref[...], axis=1, keepdims=True) + eb_ref[0, 0]
    z_ref[...] = m2 * jax.nn.sigmoid(logit)


def _edge_mlp(g, invp, winv, b1, w2, b2, ewr, ebr, ninv):
    epad = g.shape[0]
    blk = 1024
    return pl.pallas_call(
        functools.partial(_edge_mlp_body, ninv),
        grid=(epad // blk,),
        in_specs=[
            pl.BlockSpec((blk, H), lambda i: (i, 0)),
            pl.BlockSpec((blk, 8), lambda i: (i, 0)),
            pl.BlockSpec((8, H), lambda i: (0, 0)),
            pl.BlockSpec((1, H), lambda i: (0, 0)),
            pl.BlockSpec((H, H), lambda i: (0, 0)),
            pl.BlockSpec((1, H), lambda i: (0, 0)),
            pl.BlockSpec((1, H), lambda i: (0, 0)),
            pl.BlockSpec((1, H), lambda i: (0, 0)),
        ],
        out_specs=pl.BlockSpec((blk, H), lambda i: (i, 0)),
        out_shape=jax.ShapeDtypeStruct((epad, H), jnp.float32),
    )(g, invp, winv, b1, w2, b2, ewr, ebr)


ZR = 32


def _zero_fill(zrow):
    def zrowfill(r, carry):
        for k in range(H // L):
            zrow[r, pl.ds(k * L, L)] = jnp.zeros((L,), jnp.float32)
        return carry

    lax.fori_loop(0, ZR, zrowfill, 0)


def _zero_acc(acc, zrow, s):
    zpt = ACCR // NS

    def zchunk(q, carry):
        pltpu.sync_copy(zrow, acc.at[pl.ds(s * zpt + q * ZR, ZR)])
        return carry

    lax.fori_loop(0, zpt // ZR, zchunk, 0)


def _writeout(acc, out_hbm, s, nrows, out_base):
    nch = nrows // 8
    npt = (nch + NS - 1) // NS

    def wo(j, carry):
        ch = j * NS + s

        @pl.when(ch < nch)
        def _():
            pltpu.sync_copy(acc.at[pl.ds(ch * 8, 8)],
                            out_hbm.at[pl.ds(out_base + ch * 8, 8)])
        return carry

    lax.fori_loop(0, npt, wo, 0)


def _scatter_body(epad, nrec, esplit, z_hbm, idx_hbm, out_hbm,
                  acc, zv0, zv1, iv0, iv1, lbuf, zrow,
                  sl0, sl1, si0, si1, sa0, sa1):
    c = lax.axis_index("c")
    s = lax.axis_index("s")
    if esplit:
        half = nrec
        cpt = epad // (NC * NS * CH)
        tb = (c * NS + s) * cpt
        lo = c * 0
    else:
        half = nrec // 2
        cpt = epad // (NS * CH)
        tb = s * cpt
        lo = c * half
    _zero_fill(zrow)
    _zero_acc(acc, zrow, s)
    plsc.subcore_barrier()

    zvs = (zv0, zv1)
    ivs = (iv0, iv1)
    sls = (sl0, sl1)
    sis = (si0, si1)
    sas = (sa0, sa1)

    def zload(j, p):
        pltpu.async_copy(z_hbm.at[pl.ds((tb + j) * CH, CH)], zvs[p], sls[p])
        pltpu.async_copy(idx_hbm.at[pl.ds((tb + j) * CH, CH)], ivs[p], sis[p])

    def scat(j, p):
        pltpu.make_async_copy(idx_hbm.at[pl.ds(0, CH)], ivs[p],
                              sis[p]).wait()
        for k in range(CH // L):
            sl = pl.ds(k * L, L)
            t = ivs[p][sl] - lo
            ok = (t >= 0) & (t < half)
            lbuf[p, sl] = jnp.where(ok, t, half)
        pltpu.make_async_copy(z_hbm.at[pl.ds(0, CH)], zvs[p], sls[p]).wait()
        pltpu.async_copy(zvs[p], acc.at[lbuf.at[p]], sas[p], add=True)

    def wait_scat(j, p):
        pltpu.make_async_copy(zvs[p], acc.at[lbuf.at[p]], sas[p]).wait()

    zload(0, 0)
    if cpt > 1:
        zload(1, 1)
        scat(0, 0)
    npairs = (cpt - 2) // 2 if cpt > 2 else 0

    def pair(t, carry):
        j = 2 + 2 * t
        wait_scat(j - 2, 0)
        zload(j, 0)
        scat(j - 1, 1)
        wait_scat(j - 1, 1)
        zload(j + 1, 1)
        scat(j, 0)
        return carry

    lax.fori_loop(0, npairs, pair, 0)
    rem = 2 + 2 * npairs
    if cpt > 2 and rem < cpt:
        wait_scat(rem - 2, 0)
        zload(rem, 0)
        scat(rem - 1, 1)
        scat(rem, 0)
    elif cpt > 1:
        scat(cpt - 1, 1)
    else:
        scat(0, 0)
    wait_scat(0, 0)
    if cpt > 1:
        wait_scat(0, 1)
    plsc.subcore_barrier()
    if esplit:
        _writeout(acc, out_hbm, s, nrec, c * nrec)
    else:
        _writeout(acc, out_hbm, s, half, c * half)


def _scatter(z, idxp, nrec, esplit):
    epad = idxp.shape[0]
    if esplit:
        cpt = epad // (NC * NS * CH)
        out_rows = NC * nrec
    else:
        cpt = epad // (NS * CH)
        out_rows = nrec
    mesh = plsc.VectorSubcoreMesh(core_axis_name="c", subcore_axis_name="s")
    kfn = pl.kernel(
        functools.partial(_scatter_body, epad, nrec, esplit),
        out_type=jax.ShapeDtypeStruct((out_rows, H), jnp.float32),
        mesh=mesh,
        scratch_types=[
            pltpu.VMEM_SHARED((ACCR, H), jnp.float32),
            pltpu.VMEM((CH, H), jnp.float32),
            pltpu.VMEM((CH, H), jnp.float32),
            pltpu.VMEM((CH,), jnp.int32),
            pltpu.VMEM((CH,), jnp.int32),
            pltpu.VMEM((2, CH), jnp.int32),
            pltpu.VMEM((ZR, H), jnp.float32),
            pltpu.SemaphoreType.DMA,
            pltpu.SemaphoreType.DMA,
            pltpu.SemaphoreType.DMA,
            pltpu.SemaphoreType.DMA,
            pltpu.SemaphoreType.DMA,
            pltpu.SemaphoreType.DMA,
        ],
    )
    return kfn(z, idxp)


def _final0_body(u_ref, sk_ref, ma_ref, mb_ref, wm_ref, w2_ref, b1_ref,
                 be_ref, o_ref):
    m = ma_ref[...] + mb_ref[...]
    t = _silu(u_ref[...]
              + jnp.dot(m, wm_ref[...], preferred_element_type=jnp.float32)
              + b1_ref[...])
    o_ref[...] = (sk_ref[...]
                  + jnp.dot(t, w2_ref[...], preferred_element_type=jnp.float32)
                  + be_ref[...])


def _final0(u, sk, m00p, wm, w2, b1, bend):
    n = u.shape[0]
    blk = 1000
    nblk = n // blk
    return pl.pallas_call(
        _final0_body,
        grid=(nblk,),
        in_specs=[
            pl.BlockSpec((blk, H), lambda i: (i, 0)),
            pl.BlockSpec((blk, H), lambda i: (i, 0)),
            pl.BlockSpec((blk, H), lambda i: (i, 0)),
            pl.BlockSpec((blk, H), lambda i: (i + nblk, 0)),
            pl.BlockSpec((H, H), lambda i: (0, 0)),
            pl.BlockSpec((H, H), lambda i: (0, 0)),
            pl.BlockSpec((1, H), lambda i: (0, 0)),
            pl.BlockSpec((1, H), lambda i: (0, 0)),
        ],
        out_specs=pl.BlockSpec((blk, H), lambda i: (i, 0)),
        out_shape=jax.ShapeDtypeStruct((n, H), jnp.float32),
    )(u, sk, m00p, m00p, wm, w2, b1.reshape(1, H), bend.reshape(1, H))


def _final1_body(u_ref, sk_ref, m01_ref, m11_ref, wa_ref, wb_ref, w2_ref,
                 b1_ref, be_ref, o_ref):
    t = (u_ref[...]
         + jnp.dot(m01_ref[...], wa_ref[...],
                   preferred_element_type=jnp.float32)
         + jnp.dot(m11_ref[...], wb_ref[...],
                   preferred_element_type=jnp.float32))
    t = _silu(t + b1_ref[...])
    o_ref[...] = (sk_ref[...]
                  + jnp.dot(t, w2_ref[...], preferred_element_type=jnp.float32)
                  + be_ref[...])


def _final1(u, sk, m01, m11, wa, wb, w2, b1, bend):
    n = u.shape[0]
    blk = 1000
    return pl.pallas_call(
        _final1_body,
        grid=(n // blk,),
        in_specs=[
            pl.BlockSpec((blk, H), lambda i: (i, 0)),
            pl.BlockSpec((blk, H), lambda i: (i, 0)),
            pl.BlockSpec((blk, H), lambda i: (i, 0)),
            pl.BlockSpec((blk, H), lambda i: (i, 0)),
            pl.BlockSpec((H, H), lambda i: (0, 0)),
            pl.BlockSpec((H, H), lambda i: (0, 0)),
            pl.BlockSpec((H, H), lambda i: (0, 0)),
            pl.BlockSpec((1, H), lambda i: (0, 0)),
            pl.BlockSpec((1, H), lambda i: (0, 0)),
        ],
        out_specs=pl.BlockSpec((blk, H), lambda i: (i, 0)),
        out_shape=jax.ShapeDtypeStruct((n, H), jnp.float32),
    )(u, sk, m01, m11, wa, wb, w2, b1.reshape(1, H), bend.reshape(1, H))


def _message(a, b, adj, inv, mw1, mb1, mw2, mb2, ew, eb, nrec, esplit):
    e = adj.shape[1]
    ninv = inv.shape[1]
    epad = _round_up(e, NC * NS * CH)
    pad = epad - e
    i0 = adj[0].astype(jnp.int32)
    i1 = adj[1].astype(jnp.int32)
    i0p = jnp.concatenate([i0, jnp.zeros((pad,), jnp.int32)])
    i1pg = jnp.concatenate([i1, jnp.zeros((pad,), jnp.int32)])
    i1ps = jnp.concatenate([i1, jnp.full((pad,), nrec, jnp.int32)])
    invp = jnp.pad(inv, ((0, pad), (0, 8 - ninv)))
    winv = jnp.pad(mw1[2 * H:], ((0, 8 - ninv), (0, 0)))
    g = _gather_sum(a, b, i0p, i1pg)
    z = _edge_mlp(g, invp, winv, mb1.reshape(1, H), mw2, mb2.reshape(1, H),
                  ew.reshape(1, H), jnp.tile(eb.reshape(1, 1), (1, H)), ninv)
    return _scatter(z, i1ps, nrec, esplit)


def kernel(x0, x1, adj_0_0, adj_0_1, adj_1_1, inv_0_0, inv_0_1, inv_1_1,
           mw1_0_0, mb1_0_0, mw2_0_0, mb2_0_0, ew_0_0, eb_0_0,
           mw1_0_1, mb1_0_1, mw2_0_1, mb2_0_1, ew_0_1, eb_0_1,
           mw1_1_1, mb1_1_1, mw2_1_1, mb2_1_1, ew_1_1, eb_1_1,
           u0w1, u0b1, u0w2, u0b2, u1w1, u1b1, u1w2, u1b2, sw, sb):
    n0 = x0.shape[0]
    n1 = x1.shape[0]
    wcat0 = jnp.concatenate(
        [mw1_0_0[:H], mw1_0_0[H:2 * H], mw1_0_1[:H], sw, u0w1[:H]], axis=1)
    wcat1 = jnp.concatenate(
        [mw1_0_1[H:2 * H], mw1_1_1[:H], mw1_1_1[H:2 * H], sw, u1w1[:H]],
        axis=1)
    a00, b00, a01, sk0, u0 = _node_proj(x0, wcat0, 0)
    b01, a11, b11, sk1, u1 = _node_proj(x1, wcat1, 0)

    m00p = _message(a00, b00, adj_0_0, inv_0_0, mw1_0_0, mb1_0_0, mw2_0_0,
                    mb2_0_0, ew_0_0, eb_0_0, n0, True)
    m01 = _message(a01, b01, adj_0_1, inv_0_1, mw1_0_1, mb1_0_1, mw2_0_1,
                   mb2_0_1, ew_0_1, eb_0_1, n1, False)
    m11 = _message(a11, b11, adj_1_1, inv_1_1, mw1_1_1, mb1_1_1, mw2_1_1,
                   mb2_1_1, ew_1_1, eb_1_1, n1, False)

    out0 = _final0(u0, sk0, m00p, u0w1[H:], u0w2, u0b1, u0b2 + sb)
    out1 = _final1(u1, sk1, m01, m11, u1w1[H:2 * H], u1w1[2 * H:], u1w2,
                   u1b1, u1b2 + sb)
    return (out0, out1)


# speedup vs baseline: 1.2771x; 1.0622x over previous
import functools
import jax
import jax.numpy as jnp
from jax import lax
from jax.experimental import pallas as pl
from jax.experimental.pallas import tpu as pltpu
from jax.experimental.pallas import tpu_sc as plsc

H = 128
NC = 2
NS = 16
L = 16
CH = 128
ACCR = 10240


def _silu(x):
    return x * jax.nn.sigmoid(x)


def _round_up(n, m):
    return (n + m - 1) // m * m


def _proj_body(x_ref, w_ref, *out_refs):
    y = jnp.dot(x_ref[...], w_ref[...], preferred_element_type=jnp.float32)
    for i, o in enumerate(out_refs):
        o[...] = y[:, i * H:(i + 1) * H].astype(o.dtype)


def _node_proj(x, wcat, nbf16):
    n = x.shape[0]
    nout = wcat.shape[1] // H
    blk = 1000
    dts = [jnp.bfloat16] * nbf16 + [jnp.float32] * (nout - nbf16)
    return pl.pallas_call(
        _proj_body,
        grid=(n // blk,),
        in_specs=[pl.BlockSpec((blk, H), lambda i: (i, 0)),
                  pl.BlockSpec(wcat.shape, lambda i: (0, 0))],
        out_specs=[pl.BlockSpec((blk, H), lambda i: (i, 0))
                   for _ in range(nout)],
        out_shape=[jax.ShapeDtypeStruct((n, H), dt) for dt in dts],
    )(x, wcat)


ND = 3


def _gather_body(epad, a_hbm, b_hbm, i0_hbm, i1_hbm, g_hbm,
                 i0all, i1all, av0, bv0, av1, bv1, av2, bv2,
                 sg0, sg1, sg2, ss0, ss1, ss2):
    c = lax.axis_index("c")
    s = lax.axis_index("s")
    w = s * NC + c
    cpw = epad // (NC * NS * CH)
    tb = w * cpw
    pltpu.sync_copy(i0_hbm.at[pl.ds(tb * CH, cpw * CH)], i0all)
    pltpu.sync_copy(i1_hbm.at[pl.ds(tb * CH, cpw * CH)], i1all)
    avs = (av0, av1, av2)
    bvs = (bv0, bv1, bv2)
    sgs = (sg0, sg1, sg2)
    sss = (ss0, ss1, ss2)

    def issue(j, p):
        pltpu.async_copy(a_hbm.at[i0all.at[pl.ds(j * CH, CH)]], avs[p], sgs[p])
        pltpu.async_copy(b_hbm.at[i1all.at[pl.ds(j * CH, CH)]], bvs[p], sgs[p])

    def wait_store(p):
        pltpu.make_async_copy(avs[p], g_hbm.at[pl.ds(0, CH)], sss[p]).wait()

    def process(j, p):
        pltpu.make_async_copy(a_hbm.at[i0all.at[pl.ds(0, CH)]],
                              avs[p], sgs[p]).wait()
        pltpu.make_async_copy(b_hbm.at[i1all.at[pl.ds(0, CH)]],
                              bvs[p], sgs[p]).wait()
        av, bv = avs[p], bvs[p]

        def row(r, carry2):
            for k in range(H // L):
                sl = pl.ds(k * L, L)
                av[r, sl] = av[r, sl] + bv[r, sl]
            return carry2

        lax.fori_loop(0, CH, row, 0, unroll=8)
        pltpu.async_copy(av, g_hbm.at[pl.ds((tb + j) * CH, CH)], sss[p])

    def step(j, p):
        wait_store(p)
        issue(j, p)
        process(j - (ND - 1), (p - (ND - 1)) % ND)

    issue(0, 0)
    if cpw > 1:
        issue(1, 1)
    if cpw > 2:
        issue(2, 2)
        process(0, 0)
    if cpw > 3:
        wait_store(0)
        issue(3, 0)
        process(1, 1)
    ntri = (cpw - 4) // ND if cpw > 4 else 0

    def tri(t, carry):
        j = 4 + ND * t
        step(j, 4 % ND)
        step(j + 1, 5 % ND)
        step(j + 2, 6 % ND)
        return carry

    lax.fori_loop(0, ntri, tri, 0)
    for j in range(4 + ND * ntri, cpw):
        step(j, j % ND)
    if cpw > 3:
        process(cpw - 2, (cpw - 2) % ND)
        process(cpw - 1, (cpw - 1) % ND)
    elif cpw == 3:
        process(1, 1)
        process(2, 2)
    elif cpw == 2:
        process(0, 0)
        process(1, 1)
    else:
        process(0, 0)
    for p in range(min(ND, cpw)):
        wait_store(p)


def _gather_sum(a, b, i0p, i1p):
    epad = i0p.shape[0]
    cpw = epad // (NC * NS * CH)
    mesh = plsc.VectorSubcoreMesh(core_axis_name="c", subcore_axis_name="s")
    kfn = pl.kernel(
        functools.partial(_gather_body, epad),
        out_type=jax.ShapeDtypeStruct((epad, H), jnp.float32),
        mesh=mesh,
        scratch_types=[
            pltpu.VMEM((cpw * CH,), jnp.int32),
            pltpu.VMEM((cpw * CH,), jnp.int32),
            pltpu.VMEM((CH, H), jnp.float32),
            pltpu.VMEM((CH, H), jnp.float32),
            pltpu.VMEM((CH, H), jnp.float32),
            pltpu.VMEM((CH, H), jnp.float32),
            pltpu.VMEM((CH, H), jnp.float32),
            pltpu.VMEM((CH, H), jnp.float32),
            pltpu.SemaphoreType.DMA,
            pltpu.SemaphoreType.DMA,
            pltpu.SemaphoreType.DMA,
            pltpu.SemaphoreType.DMA,
            pltpu.SemaphoreType.DMA,
            pltpu.SemaphoreType.DMA,
        ],
    )
    return kfn(a, b, i0p, i1p)


def _edge_mlp_body(ninv, g_ref, inv_ref, winv_ref, b1_ref, w2_ref, b2_ref,
                   ew_ref, eb_ref, z_ref):
    pre = g_ref[...] + b1_ref[...]
    for k in range(ninv):
        pre = pre + inv_ref[:, k:k + 1] * winv_ref[k:k + 1, :]
    m = _silu(pre)
    m2 = _silu(jnp.dot(m, w2_ref[...], preferred_element_type=jnp.float32)
               + b2_ref[...])
    logit = jnp.sum(m2 * ew_ref[...], axis=1, keepdims=True) + eb_ref[0, 0]
    z_ref[...] = m2 * jax.nn.sigmoid(logit)


def _edge_mlp(g, invp, winv, b1, w2, b2, ewr, ebr, ninv):
    epad = g.shape[0]
    blk = 2048
    return pl.pallas_call(
        functools.partial(_edge_mlp_body, ninv),
        grid=(epad // blk,),
        in_specs=[
            pl.BlockSpec((blk, H), lambda i: (i, 0)),
            pl.BlockSpec((blk, 8), lambda i: (i, 0)),
            pl.BlockSpec((8, H), lambda i: (0, 0)),
            pl.BlockSpec((1, H), lambda i: (0, 0)),
            pl.BlockSpec((H, H), lambda i: (0, 0)),
            pl.BlockSpec((1, H), lambda i: (0, 0)),
            pl.BlockSpec((1, H), lambda i: (0, 0)),
            pl.BlockSpec((1, H), lambda i: (0, 0)),
        ],
        out_specs=pl.BlockSpec((blk, H), lambda i: (i, 0)),
        out_shape=jax.ShapeDtypeStruct((epad, H), jnp.float32),
    )(g, invp, winv, b1, w2, b2, ewr, ebr)


ZR = 32


def _zero_fill(zrow):
    def zrowfill(r, carry):
        for k in range(H // L):
            zrow[r, pl.ds(k * L, L)] = jnp.zeros((L,), jnp.float32)
        return carry

    lax.fori_loop(0, ZR, zrowfill, 0)


def _zero_acc(acc, zrow, s):
    zpt = ACCR // NS

    def zchunk(q, carry):
        pltpu.sync_copy(zrow, acc.at[pl.ds(s * zpt + q * ZR, ZR)])
        return carry

    lax.fori_loop(0, zpt // ZR, zchunk, 0)


def _writeout(acc, out_hbm, s, nrows, out_base):
    nch = nrows // 8
    npt = (nch + NS - 1) // NS

    def wo(j, carry):
        ch = j * NS + s

        @pl.when(ch < nch)
        def _():
            pltpu.sync_copy(acc.at[pl.ds(ch * 8, 8)],
                            out_hbm.at[pl.ds(out_base + ch * 8, 8)])
        return carry

    lax.fori_loop(0, npt, wo, 0)


def _scatter_body(epad, nrec, esplit, z_hbm, idx_hbm, out_hbm,
                  acc, zv0, zv1, iv0, iv1, lbuf, zrow,
                  sl0, sl1, si0, si1, sa0, sa1):
    c = lax.axis_index("c")
    s = lax.axis_index("s")
    if esplit:
        half = nrec
        cpt = epad // (NC * NS * CH)
        tb = (c * NS + s) * cpt
        lo = c * 0
    else:
        half = nrec // 2
        cpt = epad // (NS * CH)
        tb = s * cpt
        lo = c * half
    _zero_fill(zrow)
    _zero_acc(acc, zrow, s)
    plsc.subcore_barrier()

    zvs = (zv0, zv1)
    ivs = (iv0, iv1)
    sls = (sl0, sl1)
    sis = (si0, si1)
    sas = (sa0, sa1)

    def zload(j, p):
        pltpu.async_copy(z_hbm.at[pl.ds((tb + j) * CH, CH)], zvs[p], sls[p])
        pltpu.async_copy(idx_hbm.at[pl.ds((tb + j) * CH, CH)], ivs[p], sis[p])

    def scat(j, p):
        pltpu.make_async_copy(idx_hbm.at[pl.ds(0, CH)], ivs[p],
                              sis[p]).wait()
        for k in range(CH // L):
            sl = pl.ds(k * L, L)
            t = ivs[p][sl] - lo
            ok = (t >= 0) & (t < half)
            lbuf[p, sl] = jnp.where(ok, t, half)
        pltpu.make_async_copy(z_hbm.at[pl.ds(0, CH)], zvs[p], sls[p]).wait()
        pltpu.async_copy(zvs[p], acc.at[lbuf.at[p]], sas[p], add=True)

    def wait_scat(j, p):
        pltpu.make_async_copy(zvs[p], acc.at[lbuf.at[p]], sas[p]).wait()

    zload(0, 0)
    if cpt > 1:
        zload(1, 1)
        scat(0, 0)
    npairs = (cpt - 2) // 2 if cpt > 2 else 0

    def pair(t, carry):
        j = 2 + 2 * t
        wait_scat(j - 2, 0)
        zload(j, 0)
        scat(j - 1, 1)
        wait_scat(j - 1, 1)
        zload(j + 1, 1)
        scat(j, 0)
        return carry

    lax.fori_loop(0, npairs, pair, 0)
    rem = 2 + 2 * npairs
    if cpt > 2 and rem < cpt:
        wait_scat(rem - 2, 0)
        zload(rem, 0)
        scat(rem - 1, 1)
        scat(rem, 0)
    elif cpt > 1:
        scat(cpt - 1, 1)
    else:
        scat(0, 0)
    wait_scat(0, 0)
    if cpt > 1:
        wait_scat(0, 1)
    plsc.subcore_barrier()
    if esplit:
        _writeout(acc, out_hbm, s, nrec, c * nrec)
    else:
        _writeout(acc, out_hbm, s, half, c * half)


def _scatter(z, idxp, nrec, esplit):
    epad = idxp.shape[0]
    if esplit:
        cpt = epad // (NC * NS * CH)
        out_rows = NC * nrec
    else:
        cpt = epad // (NS * CH)
        out_rows = nrec
    mesh = plsc.VectorSubcoreMesh(core_axis_name="c", subcore_axis_name="s")
    kfn = pl.kernel(
        functools.partial(_scatter_body, epad, nrec, esplit),
        out_type=jax.ShapeDtypeStruct((out_rows, H), jnp.float32),
        mesh=mesh,
        scratch_types=[
            pltpu.VMEM_SHARED((ACCR, H), jnp.float32),
            pltpu.VMEM((CH, H), jnp.float32),
            pltpu.VMEM((CH, H), jnp.float32),
            pltpu.VMEM((CH,), jnp.int32),
            pltpu.VMEM((CH,), jnp.int32),
            pltpu.VMEM((2, CH), jnp.int32),
            pltpu.VMEM((ZR, H), jnp.float32),
            pltpu.SemaphoreType.DMA,
            pltpu.SemaphoreType.DMA,
            pltpu.SemaphoreType.DMA,
            pltpu.SemaphoreType.DMA,
            pltpu.SemaphoreType.DMA,
            pltpu.SemaphoreType.DMA,
        ],
    )
    return kfn(z, idxp)


def _final0_body(u_ref, sk_ref, ma_ref, mb_ref, wm_ref, w2_ref, b1_ref,
                 be_ref, o_ref):
    m = ma_ref[...] + mb_ref[...]
    t = _silu(u_ref[...]
              + jnp.dot(m, wm_ref[...], preferred_element_type=jnp.float32)
              + b1_ref[...])
    o_ref[...] = (sk_ref[...]
                  + jnp.dot(t, w2_ref[...], preferred_element_type=jnp.float32)
                  + be_ref[...])


def _final0(u, sk, m00p, wm, w2, b1, bend):
    n = u.shape[0]
    blk = 1000
    nblk = n // blk
    return pl.pallas_call(
        _final0_body,
        grid=(nblk,),
        in_specs=[
            pl.BlockSpec((blk, H), lambda i: (i, 0)),
            pl.BlockSpec((blk, H), lambda i: (i, 0)),
            pl.BlockSpec((blk, H), lambda i: (i, 0)),
            pl.BlockSpec((blk, H), lambda i: (i + nblk, 0)),
            pl.BlockSpec((H, H), lambda i: (0, 0)),
            pl.BlockSpec((H, H), lambda i: (0, 0)),
            pl.BlockSpec((1, H), lambda i: (0, 0)),
            pl.BlockSpec((1, H), lambda i: (0, 0)),
        ],
        out_specs=pl.BlockSpec((blk, H), lambda i: (i, 0)),
        out_shape=jax.ShapeDtypeStruct((n, H), jnp.float32),
    )(u, sk, m00p, m00p, wm, w2, b1.reshape(1, H), bend.reshape(1, H))


def _final1_body(u_ref, sk_ref, m01_ref, m11_ref, wa_ref, wb_ref, w2_ref,
                 b1_ref, be_ref, o_ref):
    t = (u_ref[...]
         + jnp.dot(m01_ref[...], wa_ref[...],
                   preferred_element_type=jnp.float32)
         + jnp.dot(m11_ref[...], wb_ref[...],
                   preferred_element_type=jnp.float32))
    t = _silu(t + b1_ref[...])
    o_ref[...] = (sk_ref[...]
                  + jnp.dot(t, w2_ref[...], preferred_element_type=jnp.float32)
                  + be_ref[...])


def _final1(u, sk, m01, m11, wa, wb, w2, b1, bend):
    n = u.shape[0]
    blk = 1000
    return pl.pallas_call(
        _final1_body,
        grid=(n // blk,),
        in_specs=[
            pl.BlockSpec((blk, H), lambda i: (i, 0)),
            pl.BlockSpec((blk, H), lambda i: (i, 0)),
            pl.BlockSpec((blk, H), lambda i: (i, 0)),
            pl.BlockSpec((blk, H), lambda i: (i, 0)),
            pl.BlockSpec((H, H), lambda i: (0, 0)),
            pl.BlockSpec((H, H), lambda i: (0, 0)),
            pl.BlockSpec((H, H), lambda i: (0, 0)),
            pl.BlockSpec((1, H), lambda i: (0, 0)),
            pl.BlockSpec((1, H), lambda i: (0, 0)),
        ],
        out_specs=pl.BlockSpec((blk, H), lambda i: (i, 0)),
        out_shape=jax.ShapeDtypeStruct((n, H), jnp.float32),
    )(u, sk, m01, m11, wa, wb, w2, b1.reshape(1, H), bend.reshape(1, H))


def _message(a, b, adj, inv, mw1, mb1, mw2, mb2, ew, eb, nrec, esplit):
    e = adj.shape[1]
    ninv = inv.shape[1]
    epad = _round_up(e, NC * NS * CH)
    pad = epad - e
    i0 = adj[0].astype(jnp.int32)
    i1 = adj[1].astype(jnp.int32)
    i0p = jnp.concatenate([i0, jnp.zeros((pad,), jnp.int32)])
    i1pg = jnp.concatenate([i1, jnp.zeros((pad,), jnp.int32)])
    i1ps = jnp.concatenate([i1, jnp.full((pad,), nrec, jnp.int32)])
    invp = jnp.pad(inv, ((0, pad), (0, 8 - ninv)))
    winv = jnp.pad(mw1[2 * H:], ((0, 8 - ninv), (0, 0)))
    g = _gather_sum(a, b, i0p, i1pg)
    z = _edge_mlp(g, invp, winv, mb1.reshape(1, H), mw2, mb2.reshape(1, H),
                  ew.reshape(1, H), jnp.tile(eb.reshape(1, 1), (1, H)), ninv)
    return _scatter(z, i1ps, nrec, esplit)


def kernel(x0, x1, adj_0_0, adj_0_1, adj_1_1, inv_0_0, inv_0_1, inv_1_1,
           mw1_0_0, mb1_0_0, mw2_0_0, mb2_0_0, ew_0_0, eb_0_0,
           mw1_0_1, mb1_0_1, mw2_0_1, mb2_0_1, ew_0_1, eb_0_1,
           mw1_1_1, mb1_1_1, mw2_1_1, mb2_1_1, ew_1_1, eb_1_1,
           u0w1, u0b1, u0w2, u0b2, u1w1, u1b1, u1w2, u1b2, sw, sb):
    n0 = x0.shape[0]
    n1 = x1.shape[0]
    wcat0 = jnp.concatenate(
        [mw1_0_0[:H], mw1_0_0[H:2 * H], mw1_0_1[:H], sw, u0w1[:H]], axis=1)
    wcat1 = jnp.concatenate(
        [mw1_0_1[H:2 * H], mw1_1_1[:H], mw1_1_1[H:2 * H], sw, u1w1[:H]],
        axis=1)
    a00, b00, a01, sk0, u0 = _node_proj(x0, wcat0, 0)
    b01, a11, b11, sk1, u1 = _node_proj(x1, wcat1, 0)

    m00p = _message(a00, b00, adj_0_0, inv_0_0, mw1_0_0, mb1_0_0, mw2_0_0,
                    mb2_0_0, ew_0_0, eb_0_0, n0, True)
    m01 = _message(a01, b01, adj_0_1, inv_0_1, mw1_0_1, mb1_0_1, mw2_0_1,
                   mb2_0_1, ew_0_1, eb_0_1, n1, False)
    m11 = _message(a11, b11, adj_1_1, inv_1_1, mw1_1_1, mb1_1_1, mw2_1_1,
                   mb2_1_1, ew_1_1, eb_1_1, n1, False)

    out0 = _final0(u0, sk0, m00p, u0w1[H:], u0w2, u0b1, u0b2 + sb)
    out1 = _final1(u1, sk1, m01, m11, u1w1[H:2 * H], u1w1[2 * H:], u1w2,
                   u1b1, u1b2 + sb)
    return (out0, out1)


# speedup vs baseline: 1.2931x; 1.0125x over previous
import functools
import jax
import jax.numpy as jnp
from jax import lax
from jax.experimental import pallas as pl
from jax.experimental.pallas import tpu as pltpu
from jax.experimental.pallas import tpu_sc as plsc

H = 128
NC = 2
NS = 16
L = 16
CH = 128
ACCR = 10240


def _silu(x):
    return x * jax.nn.sigmoid(x)


def _round_up(n, m):
    return (n + m - 1) // m * m


def _proj_body(x_ref, w_ref, *out_refs):
    y = jnp.dot(x_ref[...], w_ref[...], preferred_element_type=jnp.float32)
    for i, o in enumerate(out_refs):
        o[...] = y[:, i * H:(i + 1) * H].astype(o.dtype)


def _node_proj(x, wcat, nbf16):
    n = x.shape[0]
    nout = wcat.shape[1] // H
    blk = 1000
    dts = [jnp.bfloat16] * nbf16 + [jnp.float32] * (nout - nbf16)
    return pl.pallas_call(
        _proj_body,
        grid=(n // blk,),
        in_specs=[pl.BlockSpec((blk, H), lambda i: (i, 0)),
                  pl.BlockSpec(wcat.shape, lambda i: (0, 0))],
        out_specs=[pl.BlockSpec((blk, H), lambda i: (i, 0))
                   for _ in range(nout)],
        out_shape=[jax.ShapeDtypeStruct((n, H), dt) for dt in dts],
    )(x, wcat)


ND = 3


def _gather_body(epad, a_hbm, b_hbm, i0_hbm, i1_hbm, g_hbm,
                 i0all, i1all, av0, bv0, av1, bv1, av2, bv2,
                 sg0, sg1, sg2, ss0, ss1, ss2):
    c = lax.axis_index("c")
    s = lax.axis_index("s")
    w = s * NC + c
    cpw = epad // (NC * NS * CH)
    tb = w * cpw
    pltpu.sync_copy(i0_hbm.at[pl.ds(tb * CH, cpw * CH)], i0all)
    pltpu.sync_copy(i1_hbm.at[pl.ds(tb * CH, cpw * CH)], i1all)
    avs = (av0, av1, av2)
    bvs = (bv0, bv1, bv2)
    sgs = (sg0, sg1, sg2)
    sss = (ss0, ss1, ss2)

    def issue(j, p):
        pltpu.async_copy(a_hbm.at[i0all.at[pl.ds(j * CH, CH)]], avs[p], sgs[p])
        pltpu.async_copy(b_hbm.at[i1all.at[pl.ds(j * CH, CH)]], bvs[p], sgs[p])

    def wait_store(p):
        pltpu.make_async_copy(avs[p], g_hbm.at[pl.ds(0, CH)], sss[p]).wait()

    def process(j, p):
        pltpu.make_async_copy(a_hbm.at[i0all.at[pl.ds(0, CH)]],
                              avs[p], sgs[p]).wait()
        pltpu.make_async_copy(b_hbm.at[i1all.at[pl.ds(0, CH)]],
                              bvs[p], sgs[p]).wait()
        av, bv = avs[p], bvs[p]

        def row(r, carry2):
            for k in range(H // L):
                sl = pl.ds(k * L, L)
                av[r, sl] = av[r, sl] + bv[r, sl]
            return carry2

        lax.fori_loop(0, CH, row, 0, unroll=8)
        pltpu.async_copy(av, g_hbm.at[pl.ds((tb + j) * CH, CH)], sss[p])

    def step(j, p):
        wait_store(p)
        issue(j, p)
        process(j - (ND - 1), (p - (ND - 1)) % ND)

    issue(0, 0)
    if cpw > 1:
        issue(1, 1)
    if cpw > 2:
        issue(2, 2)
        process(0, 0)
    if cpw > 3:
        wait_store(0)
        issue(3, 0)
        process(1, 1)
    ntri = (cpw - 4) // ND if cpw > 4 else 0

    def tri(t, carry):
        j = 4 + ND * t
        step(j, 4 % ND)
        step(j + 1, 5 % ND)
        step(j + 2, 6 % ND)
        return carry

    lax.fori_loop(0, ntri, tri, 0)
    for j in range(4 + ND * ntri, cpw):
        step(j, j % ND)
    if cpw > 3:
        process(cpw - 2, (cpw - 2) % ND)
        process(cpw - 1, (cpw - 1) % ND)
    elif cpw == 3:
        process(1, 1)
        process(2, 2)
    elif cpw == 2:
        process(0, 0)
        process(1, 1)
    else:
        process(0, 0)
    for p in range(min(ND, cpw)):
        wait_store(p)


def _gather_sum(a, b, i0p, i1p):
    epad = i0p.shape[0]
    cpw = epad // (NC * NS * CH)
    mesh = plsc.VectorSubcoreMesh(core_axis_name="c", subcore_axis_name="s")
    kfn = pl.kernel(
        functools.partial(_gather_body, epad),
        out_type=jax.ShapeDtypeStruct((epad, H), jnp.float32),
        mesh=mesh,
        scratch_types=[
            pltpu.VMEM((cpw * CH,), jnp.int32),
            pltpu.VMEM((cpw * CH,), jnp.int32),
            pltpu.VMEM((CH, H), jnp.float32),
            pltpu.VMEM((CH, H), jnp.float32),
            pltpu.VMEM((CH, H), jnp.float32),
            pltpu.VMEM((CH, H), jnp.float32),
            pltpu.VMEM((CH, H), jnp.float32),
            pltpu.VMEM((CH, H), jnp.float32),
            pltpu.SemaphoreType.DMA,
            pltpu.SemaphoreType.DMA,
            pltpu.SemaphoreType.DMA,
            pltpu.SemaphoreType.DMA,
            pltpu.SemaphoreType.DMA,
            pltpu.SemaphoreType.DMA,
        ],
    )
    return kfn(a, b, i0p, i1p)


def _edge_mlp_body(ninv, g_ref, inv_ref, winv_ref, b1_ref, w2_ref, b2_ref,
                   ew_ref, eb_ref, z_ref):
    pre = g_ref[...] + b1_ref[...]
    for k in range(ninv):
        pre = pre + inv_ref[:, k:k + 1] * winv_ref[k:k + 1, :]
    m = _silu(pre)
    m2 = _silu(jnp.dot(m, w2_ref[...], preferred_element_type=jnp.float32)
               + b2_ref[...])
    logit = jnp.sum(m2 * ew_ref[...], axis=1, keepdims=True) + eb_ref[0, 0]
    z_ref[...] = m2 * jax.nn.sigmoid(logit)


def _edge_mlp(g, invp, winv, b1, w2, b2, ewr, ebr, ninv):
    epad = g.shape[0]
    blk = 4096
    return pl.pallas_call(
        functools.partial(_edge_mlp_body, ninv),
        grid=(epad // blk,),
        in_specs=[
            pl.BlockSpec((blk, H), lambda i: (i, 0)),
            pl.BlockSpec((blk, 8), lambda i: (i, 0)),
            pl.BlockSpec((8, H), lambda i: (0, 0)),
            pl.BlockSpec((1, H), lambda i: (0, 0)),
            pl.BlockSpec((H, H), lambda i: (0, 0)),
            pl.BlockSpec((1, H), lambda i: (0, 0)),
            pl.BlockSpec((1, H), lambda i: (0, 0)),
            pl.BlockSpec((1, H), lambda i: (0, 0)),
        ],
        out_specs=pl.BlockSpec((blk, H), lambda i: (i, 0)),
        out_shape=jax.ShapeDtypeStruct((epad, H), jnp.float32),
    )(g, invp, winv, b1, w2, b2, ewr, ebr)


ZR = 32


def _zero_fill(zrow):
    def zrowfill(r, carry):
        for k in range(H // L):
            zrow[r, pl.ds(k * L, L)] = jnp.zeros((L,), jnp.float32)
        return carry

    lax.fori_loop(0, ZR, zrowfill, 0)


def _zero_acc(acc, zrow, s):
    zpt = ACCR // NS

    def zchunk(q, carry):
        pltpu.sync_copy(zrow, acc.at[pl.ds(s * zpt + q * ZR, ZR)])
        return carry

    lax.fori_loop(0, zpt // ZR, zchunk, 0)


def _writeout(acc, out_hbm, s, nrows, out_base):
    nch = nrows // 8
    npt = (nch + NS - 1) // NS

    def wo(j, carry):
        ch = j * NS + s

        @pl.when(ch < nch)
        def _():
            pltpu.sync_copy(acc.at[pl.ds(ch * 8, 8)],
                            out_hbm.at[pl.ds(out_base + ch * 8, 8)])
        return carry

    lax.fori_loop(0, npt, wo, 0)


def _scatter_body(epad, nrec, esplit, z_hbm, idx_hbm, out_hbm,
                  acc, zv0, zv1, iv0, iv1, lbuf, zrow,
                  sl0, sl1, si0, si1, sa0, sa1):
    c = lax.axis_index("c")
    s = lax.axis_index("s")
    if esplit:
        half = nrec
        cpt = epad // (NC * NS * CH)
        tb = (c * NS + s) * cpt
        lo = c * 0
    else:
        half = nrec // 2
        cpt = epad // (NS * CH)
        tb = s * cpt
        lo = c * half
    _zero_fill(zrow)
    _zero_acc(acc, zrow, s)
    plsc.subcore_barrier()

    zvs = (zv0, zv1)
    ivs = (iv0, iv1)
    sls = (sl0, sl1)
    sis = (si0, si1)
    sas = (sa0, sa1)

    def zload(j, p):
        pltpu.async_copy(z_hbm.at[pl.ds((tb + j) * CH, CH)], zvs[p], sls[p])
        pltpu.async_copy(idx_hbm.at[pl.ds((tb + j) * CH, CH)], ivs[p], sis[p])

    def scat(j, p):
        pltpu.make_async_copy(idx_hbm.at[pl.ds(0, CH)], ivs[p],
                              sis[p]).wait()
        for k in range(CH // L):
            sl = pl.ds(k * L, L)
            t = ivs[p][sl] - lo
            ok = (t >= 0) & (t < half)
            lbuf[p, sl] = jnp.where(ok, t, half)
        pltpu.make_async_copy(z_hbm.at[pl.ds(0, CH)], zvs[p], sls[p]).wait()
        pltpu.async_copy(zvs[p], acc.at[lbuf.at[p]], sas[p], add=True)

    def wait_scat(j, p):
        pltpu.make_async_copy(zvs[p], acc.at[lbuf.at[p]], sas[p]).wait()

    zload(0, 0)
    if cpt > 1:
        zload(1, 1)
        scat(0, 0)
    npairs = (cpt - 2) // 2 if cpt > 2 else 0

    def pair(t, carry):
        j = 2 + 2 * t
        wait_scat(j - 2, 0)
        zload(j, 0)
        scat(j - 1, 1)
        wait_scat(j - 1, 1)
        zload(j + 1, 1)
        scat(j, 0)
        return carry

    lax.fori_loop(0, npairs, pair, 0)
    rem = 2 + 2 * npairs
    if cpt > 2 and rem < cpt:
        wait_scat(rem - 2, 0)
        zload(rem, 0)
        scat(rem - 1, 1)
        scat(rem, 0)
    elif cpt > 1:
        scat(cpt - 1, 1)
    else:
        scat(0, 0)
    wait_scat(0, 0)
    if cpt > 1:
        wait_scat(0, 1)
    plsc.subcore_barrier()
    if esplit:
        _writeout(acc, out_hbm, s, nrec, c * nrec)
    else:
        _writeout(acc, out_hbm, s, half, c * half)


def _scatter(z, idxp, nrec, esplit):
    epad = idxp.shape[0]
    if esplit:
        cpt = epad // (NC * NS * CH)
        out_rows = NC * nrec
    else:
        cpt = epad // (NS * CH)
        out_rows = nrec
    mesh = plsc.VectorSubcoreMesh(core_axis_name="c", subcore_axis_name="s")
    kfn = pl.kernel(
        functools.partial(_scatter_body, epad, nrec, esplit),
        out_type=jax.ShapeDtypeStruct((out_rows, H), jnp.float32),
        mesh=mesh,
        scratch_types=[
            pltpu.VMEM_SHARED((ACCR, H), jnp.float32),
            pltpu.VMEM((CH, H), jnp.float32),
            pltpu.VMEM((CH, H), jnp.float32),
            pltpu.VMEM((CH,), jnp.int32),
            pltpu.VMEM((CH,), jnp.int32),
            pltpu.VMEM((2, CH), jnp.int32),
            pltpu.VMEM((ZR, H), jnp.float32),
            pltpu.SemaphoreType.DMA,
            pltpu.SemaphoreType.DMA,
            pltpu.SemaphoreType.DMA,
            pltpu.SemaphoreType.DMA,
            pltpu.SemaphoreType.DMA,
            pltpu.SemaphoreType.DMA,
        ],
    )
    return kfn(z, idxp)


def _final0_body(u_ref, sk_ref, ma_ref, mb_ref, wm_ref, w2_ref, b1_ref,
                 be_ref, o_ref):
    m = ma_ref[...] + mb_ref[...]
    t = _silu(u_ref[...]
              + jnp.dot(m, wm_ref[...], preferred_element_type=jnp.float32)
              + b1_ref[...])
    o_ref[...] = (sk_ref[...]
                  + jnp.dot(t, w2_ref[...], preferred_element_type=jnp.float32)
                  + be_ref[...])


def _final0(u, sk, m00p, wm, w2, b1, bend):
    n = u.shape[0]
    blk = 1000
    nblk = n // blk
    return pl.pallas_call(
        _final0_body,
        grid=(nblk,),
        in_specs=[
            pl.BlockSpec((blk, H), lambda i: (i, 0)),
            pl.BlockSpec((blk, H), lambda i: (i, 0)),
            pl.BlockSpec((blk, H), lambda i: (i, 0)),
            pl.BlockSpec((blk, H), lambda i: (i + nblk, 0)),
            pl.BlockSpec((H, H), lambda i: (0, 0)),
            pl.BlockSpec((H, H), lambda i: (0, 0)),
            pl.BlockSpec((1, H), lambda i: (0, 0)),
            pl.BlockSpec((1, H), lambda i: (0, 0)),
        ],
        out_specs=pl.BlockSpec((blk, H), lambda i: (i, 0)),
        out_shape=jax.ShapeDtypeStruct((n, H), jnp.float32),
    )(u, sk, m00p, m00p, wm, w2, b1.reshape(1, H), bend.reshape(1, H))


def _final1_body(u_ref, sk_ref, m01_ref, m11_ref, wa_ref, wb_ref, w2_ref,
                 b1_ref, be_ref, o_ref):
    t = (u_ref[...]
         + jnp.dot(m01_ref[...], wa_ref[...],
                   preferred_element_type=jnp.float32)
         + jnp.dot(m11_ref[...], wb_ref[...],
                   preferred_element_type=jnp.float32))
    t = _silu(t + b1_ref[...])
    o_ref[...] = (sk_ref[...]
                  + jnp.dot(t, w2_ref[...], preferred_element_type=jnp.float32)
                  + be_ref[...])


def _final1(u, sk, m01, m11, wa, wb, w2, b1, bend):
    n = u.shape[0]
    blk = 1000
    return pl.pallas_call(
        _final1_body,
        grid=(n // blk,),
        in_specs=[
            pl.BlockSpec((blk, H), lambda i: (i, 0)),
            pl.BlockSpec((blk, H), lambda i: (i, 0)),
            pl.BlockSpec((blk, H), lambda i: (i, 0)),
            pl.BlockSpec((blk, H), lambda i: (i, 0)),
            pl.BlockSpec((H, H), lambda i: (0, 0)),
            pl.BlockSpec((H, H), lambda i: (0, 0)),
            pl.BlockSpec((H, H), lambda i: (0, 0)),
            pl.BlockSpec((1, H), lambda i: (0, 0)),
            pl.BlockSpec((1, H), lambda i: (0, 0)),
        ],
        out_specs=pl.BlockSpec((blk, H), lambda i: (i, 0)),
        out_shape=jax.ShapeDtypeStruct((n, H), jnp.float32),
    )(u, sk, m01, m11, wa, wb, w2, b1.reshape(1, H), bend.reshape(1, H))


def _message(a, b, adj, inv, mw1, mb1, mw2, mb2, ew, eb, nrec, esplit):
    e = adj.shape[1]
    ninv = inv.shape[1]
    epad = _round_up(e, NC * NS * CH)
    pad = epad - e
    i0 = adj[0].astype(jnp.int32)
    i1 = adj[1].astype(jnp.int32)
    i0p = jnp.concatenate([i0, jnp.zeros((pad,), jnp.int32)])
    i1pg = jnp.concatenate([i1, jnp.zeros((pad,), jnp.int32)])
    i1ps = jnp.concatenate([i1, jnp.full((pad,), nrec, jnp.int32)])
    invp = jnp.pad(inv, ((0, pad), (0, 8 - ninv)))
    winv = jnp.pad(mw1[2 * H:], ((0, 8 - ninv), (0, 0)))
    g = _gather_sum(a, b, i0p, i1pg)
    z = _edge_mlp(g, invp, winv, mb1.reshape(1, H), mw2, mb2.reshape(1, H),
                  ew.reshape(1, H), jnp.tile(eb.reshape(1, 1), (1, H)), ninv)
    return _scatter(z, i1ps, nrec, esplit)


def kernel(x0, x1, adj_0_0, adj_0_1, adj_1_1, inv_0_0, inv_0_1, inv_1_1,
           mw1_0_0, mb1_0_0, mw2_0_0, mb2_0_0, ew_0_0, eb_0_0,
           mw1_0_1, mb1_0_1, mw2_0_1, mb2_0_1, ew_0_1, eb_0_1,
           mw1_1_1, mb1_1_1, mw2_1_1, mb2_1_1, ew_1_1, eb_1_1,
           u0w1, u0b1, u0w2, u0b2, u1w1, u1b1, u1w2, u1b2, sw, sb):
    n0 = x0.shape[0]
    n1 = x1.shape[0]
    wcat0 = jnp.concatenate(
        [mw1_0_0[:H], mw1_0_0[H:2 * H], mw1_0_1[:H], sw, u0w1[:H]], axis=1)
    wcat1 = jnp.concatenate(
        [mw1_0_1[H:2 * H], mw1_1_1[:H], mw1_1_1[H:2 * H], sw, u1w1[:H]],
        axis=1)
    a00, b00, a01, sk0, u0 = _node_proj(x0, wcat0, 0)
    b01, a11, b11, sk1, u1 = _node_proj(x1, wcat1, 0)

    m00p = _message(a00, b00, adj_0_0, inv_0_0, mw1_0_0, mb1_0_0, mw2_0_0,
                    mb2_0_0, ew_0_0, eb_0_0, n0, True)
    m01 = _message(a01, b01, adj_0_1, inv_0_1, mw1_0_1, mb1_0_1, mw2_0_1,
                   mb2_0_1, ew_0_1, eb_0_1, n1, False)
    m11 = _message(a11, b11, adj_1_1, inv_1_1, mw1_1_1, mb1_1_1, mw2_1_1,
                   mb2_1_1, ew_1_1, eb_1_1, n1, False)

    out0 = _final0(u0, sk0, m00p, u0w1[H:], u0w2, u0b1, u0b2 + sb)
    out1 = _final1(u1, sk1, m01, m11, u1w1[H:2 * H], u1w1[2 * H:], u1w2,
                   u1b1, u1b2 + sb)
    return (out0, out1)
